# Initial kernel scaffold; baseline (speedup 1.0000x reference)
import jax
import jax.numpy as jnp
from jax.experimental import pallas as pl


def kernel(x, edge_index, edge_attr, We0, be0, We1, be1, We2, be2, Wi0, bi0, Wi1, bi1, Wi2, bi2, Wn0, bn0, Wn1, bn1, Wn2, bn2, Wo0, bo0, Wo1, bo1, Wo2, bo2):
    raise NotImplementedError("write your pallas kernel here")


# speedup vs baseline: 2.0898x; 2.0898x over previous
import functools

import jax
import jax.numpy as jnp
from jax import lax
from jax.experimental import pallas as pl
from jax.experimental.pallas import tpu as pltpu
from jax.experimental.pallas import tpu_sc as plsc

N = 10000
E = 320000
D_IN = 128
D_F = 128
D_H = 256
D_E = 16
NCLS = 10

NUM_SC = 2
NUM_TILES = 16
NW = NUM_SC * NUM_TILES
EPW = E // NW
ECH = 80
NCH = EPW // ECH
NPAD = 10240
ROWS_PER_TILE = NPAD // NUM_TILES

NBLK = 1000
EBLK = 512

_F32 = jnp.float32


def _mm(a, b):
    return jnp.dot(a, b, preferred_element_type=jnp.float32)


def _encoder_body(x_ref, w0, b0, w1, b1, w2, b2, h_ref):
    t = jax.nn.relu(_mm(x_ref[...], w0[...]) + b0[...])
    t = jax.nn.relu(_mm(t, w1[...]) + b1[...])
    h_ref[...] = _mm(t, w2[...]) + b2[...]


def _encode(x, We0, be0, We1, be1, We2, be2):
    full = lambda a: pl.BlockSpec(a.shape, lambda i: (0,) * a.ndim)
    return pl.pallas_call(
        _encoder_body,
        grid=(N // NBLK,),
        in_specs=[pl.BlockSpec((NBLK, D_IN), lambda i: (i, 0)),
                  full(We0), full(be0), full(We1), full(be1), full(We2), full(be2)],
        out_specs=pl.BlockSpec((NBLK, D_F), lambda i: (i, 0)),
        out_shape=jax.ShapeDtypeStruct((N, D_F), _F32),
    )(x, We0, be0, We1, be1, We2, be2)


def _sc_gather(h, src, dst):
    mesh = plsc.VectorSubcoreMesh(core_axis_name="c", subcore_axis_name="s")

    @functools.partial(
        pl.kernel,
        out_type=[jax.ShapeDtypeStruct((E, D_F), _F32),
                  jax.ShapeDtypeStruct((E, D_F), _F32)],
        mesh=mesh,
        scratch_types=[pltpu.VMEM((ECH,), jnp.int32),
                       pltpu.VMEM((ECH,), jnp.int32),
                       pltpu.VMEM((ECH, D_F), _F32),
                       pltpu.VMEM((ECH, D_F), _F32),
                       pltpu.SemaphoreType.DMA,
                       pltpu.SemaphoreType.DMA],
    )
    def k(h_hbm, src_hbm, dst_hbm, gs_hbm, gd_hbm, si_v, di_v, sr_v, dr_v, s_sem, d_sem):
        wid = lax.axis_index("s") * NUM_SC + lax.axis_index("c")
        base = wid * EPW

        @pl.loop(0, NCH)
        def _(i):
            off = base + i * ECH
            pltpu.sync_copy(src_hbm.at[pl.ds(off, ECH)], si_v)
            pltpu.sync_copy(dst_hbm.at[pl.ds(off, ECH)], di_v)
            cs = pltpu.async_copy(h_hbm.at[si_v], sr_v, s_sem)
            cd = pltpu.async_copy(h_hbm.at[di_v], dr_v, d_sem)
            cs.wait()
            cd.wait()
            pltpu.sync_copy(sr_v, gs_hbm.at[pl.ds(off, ECH)])
            pltpu.sync_copy(dr_v, gd_hbm.at[pl.ds(off, ECH)])

    return k(h, src, dst)


def _edge_body(gs_ref, gd_ref, ea_ref, w0s, w0d, w0e, b0, w1, b1, w2, b2, m_ref):
    t = (_mm(gs_ref[...], w0s[...]) + _mm(gd_ref[...], w0d[...])
         + _mm(ea_ref[...], w0e[...]) + b0[...])
    t = jax.nn.relu(t)
    t = jax.nn.relu(_mm(t, w1[...]) + b1[...])
    m_ref[...] = _mm(t, w2[...]) + b2[...]


def _edge_mlp(gs, gd, ea, Wi0s, Wi0d, Wi0e, bi0, Wi1, bi1, Wi2, bi2):
    full = lambda a: pl.BlockSpec(a.shape, lambda i: (0,) * a.ndim)
    return pl.pallas_call(
        _edge_body,
        grid=(E // EBLK,),
        in_specs=[pl.BlockSpec((EBLK, D_F), lambda i: (i, 0)),
                  pl.BlockSpec((EBLK, D_F), lambda i: (i, 0)),
                  pl.BlockSpec((EBLK, D_E), lambda i: (i, 0)),
                  full(Wi0s), full(Wi0d), full(Wi0e), full(bi0),
                  full(Wi1), full(bi1), full(Wi2), full(bi2)],
        out_specs=pl.BlockSpec((EBLK, D_F), lambda i: (i, 0)),
        out_shape=jax.ShapeDtypeStruct((E, D_F), _F32),
    )(gs, gd, ea, Wi0s, Wi0d, Wi0e, bi0, Wi1, bi1, Wi2, bi2)


def _sc_segsum(m, dst):
    mesh = plsc.VectorSubcoreMesh(core_axis_name="c", subcore_axis_name="s")

    @functools.partial(
        pl.kernel,
        out_type=jax.ShapeDtypeStruct((NUM_SC * NPAD, D_F), _F32),
        mesh=mesh,
        scratch_types=[pltpu.VMEM((ECH, D_F), _F32),
                       pltpu.VMEM((ECH,), jnp.int32),
                       pltpu.VMEM_SHARED((NPAD, D_F), _F32)],
    )
    def agg_k(m_hbm, dst_hbm, agg_hbm, m_v, idx_v, sh_agg):
        cid = lax.axis_index("c")
        sid = lax.axis_index("s")
        wid = sid * NUM_SC + cid

        @pl.loop(0, ECH)
        def _(r):
            @pl.loop(0, D_F, step=16)
            def _(c):
                m_v.at[r, pl.ds(c, 16)][...] = jnp.zeros((16,), _F32)

        @pl.loop(0, ROWS_PER_TILE // ECH)
        def _(j):
            pltpu.sync_copy(m_v, sh_agg.at[pl.ds(sid * ROWS_PER_TILE + j * ECH, ECH)])

        plsc.subcore_barrier()
        base = wid * EPW

        @pl.loop(0, NCH)
        def _(i):
            off = base + i * ECH
            pltpu.sync_copy(m_hbm.at[pl.ds(off, ECH)], m_v)
            pltpu.sync_copy(dst_hbm.at[pl.ds(off, ECH)], idx_v)
            pltpu.sync_copy(m_v, sh_agg.at[idx_v], add=True)

        plsc.subcore_barrier()
        r0 = sid * ROWS_PER_TILE
        pltpu.sync_copy(sh_agg.at[pl.ds(r0, ROWS_PER_TILE)],
                        agg_hbm.at[pl.ds(cid * NPAD + r0, ROWS_PER_TILE)])

    @functools.partial(
        pl.kernel,
        out_type=jax.ShapeDtypeStruct((NUM_SC * NPAD, D_F), _F32),
        mesh=mesh,
        scratch_types=[pltpu.VMEM((ECH, D_F), _F32),
                       pltpu.VMEM((ECH,), jnp.int32),
                       pltpu.VMEM_SHARED((NPAD, D_F), _F32)],
    )
    def cnt_k(dst_hbm, cnt_hbm, ones_v, idx_v, sh_cnt):
        cid = lax.axis_index("c")
        sid = lax.axis_index("s")
        wid = sid * NUM_SC + cid

        @pl.loop(0, ECH)
        def _(r):
            @pl.loop(0, D_F, step=16)
            def _(c):
                ones_v.at[r, pl.ds(c, 16)][...] = jnp.zeros((16,), _F32)

        @pl.loop(0, ROWS_PER_TILE // ECH)
        def _(j):
            pltpu.sync_copy(ones_v, sh_cnt.at[pl.ds(sid * ROWS_PER_TILE + j * ECH, ECH)])

        @pl.loop(0, ECH)
        def _(r):
            @pl.loop(0, D_F, step=16)
            def _(c):
                ones_v.at[r, pl.ds(c, 16)][...] = jnp.full((16,), 1.0, _F32)

        plsc.subcore_barrier()
        base = wid * EPW

        @pl.loop(0, NCH)
        def _(i):
            off = base + i * ECH
            pltpu.sync_copy(dst_hbm.at[pl.ds(off, ECH)], idx_v)
            pltpu.sync_copy(ones_v, sh_cnt.at[idx_v], add=True)

        plsc.subcore_barrier()
        r0 = sid * ROWS_PER_TILE
        pltpu.sync_copy(sh_cnt.at[pl.ds(r0, ROWS_PER_TILE)],
                        cnt_hbm.at[pl.ds(cid * NPAD + r0, ROWS_PER_TILE)])

    return agg_k(m, dst), cnt_k(dst)


def _node_body(h_ref, a0_ref, a1_ref, c0_ref, c1_ref, wn0h, wn0a, bn0, wn1, bn1,
               wn2, bn2, wo0, bo0, wo1, bo1, wo2, bo2, out_ref):
    cnt = c0_ref[...][:, 0:1] + c1_ref[...][:, 0:1]
    agg = (a0_ref[...] + a1_ref[...]) / jnp.maximum(cnt, 1.0)
    t = jax.nn.relu(_mm(h_ref[...], wn0h[...]) + _mm(agg, wn0a[...]) + bn0[...])
    t = jax.nn.relu(_mm(t, wn1[...]) + bn1[...])
    h2 = _mm(t, wn2[...]) + bn2[...]
    t = jax.nn.relu(_mm(h2, wo0[...]) + bo0[...])
    t = jax.nn.relu(_mm(t, wo1[...]) + bo1[...])
    out_ref[...] = _mm(t, wo2[...]) + bo2[...]


def _node_mlp(h, a0, a1, c0, c1, Wn0h, Wn0a, bn0, Wn1, bn1, Wn2, bn2,
              Wo0, bo0, Wo1, bo1, Wo2, bo2):
    full = lambda a: pl.BlockSpec(a.shape, lambda i: (0,) * a.ndim)
    row = lambda d: pl.BlockSpec((NBLK, d), lambda i: (i, 0))
    ws = (Wn0h, Wn0a, bn0, Wn1, bn1, Wn2, bn2, Wo0, bo0, Wo1, bo1, Wo2, bo2)
    return pl.pallas_call(
        _node_body,
        grid=(N // NBLK,),
        in_specs=[row(D_F), row(D_F), row(D_F), row(D_F), row(D_F)]
                 + [full(w) for w in ws],
        out_specs=pl.BlockSpec((NBLK, NCLS), lambda i: (i, 0)),
        out_shape=jax.ShapeDtypeStruct((N, NCLS), _F32),
    )(h, a0, a1, c0, c1, *ws)


def kernel(x, edge_index, edge_attr, We0, be0, We1, be1, We2, be2,
           Wi0, bi0, Wi1, bi1, Wi2, bi2, Wn0, bn0, Wn1, bn1, Wn2, bn2,
           Wo0, bo0, Wo1, bo1, Wo2, bo2):
    src = edge_index[0]
    dst = edge_index[1]
    r1 = lambda b: b.reshape(1, -1)

    h = _encode(x, We0, r1(be0), We1, r1(be1), We2, r1(be2))
    gs, gd = _sc_gather(h, src, dst)
    m = _edge_mlp(gs, gd, edge_attr,
                  Wi0[:D_F], Wi0[D_F:2 * D_F], Wi0[2 * D_F:], r1(bi0),
                  Wi1, r1(bi1), Wi2, r1(bi2))
    agg_flat, cnt_flat = _sc_segsum(m, dst)
    aggp = agg_flat.reshape(NUM_SC, NPAD, D_F)[:, :N]
    cntp = cnt_flat.reshape(NUM_SC, NPAD, D_F)[:, :N]
    out = _node_mlp(h, aggp[0], aggp[1], cntp[0], cntp[1],
                    Wn0[:D_F], Wn0[D_F:], r1(bn0), Wn1, r1(bn1), Wn2, r1(bn2),
                    Wo0, r1(bo0), Wo1, r1(bo1), Wo2, r1(bo2))
    return out


# speedup vs baseline: 2.0952x; 1.0026x over previous
import functools

import jax
import jax.numpy as jnp
from jax import lax
from jax.experimental import pallas as pl
from jax.experimental.pallas import tpu as pltpu
from jax.experimental.pallas import tpu_sc as plsc

N = 10000
E = 320000
D_IN = 128
D_F = 128
D_H = 256
D_E = 16
NCLS = 10

NUM_SC = 2
NUM_TILES = 16
NW = NUM_SC * NUM_TILES
EPW = E // NW
ECH = 80
NCH = EPW // ECH
NPAD = 10240
ROWS_PER_TILE = NPAD // NUM_TILES

NBLK = 1000
EBLK = 512

_F32 = jnp.float32


def _mm(a, b):
    return jnp.dot(a, b, preferred_element_type=jnp.float32)


def _encoder_body(x_ref, w0, b0, w1, b1, w2, b2, h_ref):
    t = jax.nn.relu(_mm(x_ref[...], w0[...]) + b0[...])
    t = jax.nn.relu(_mm(t, w1[...]) + b1[...])
    h_ref[...] = _mm(t, w2[...]) + b2[...]


def _encode(x, We0, be0, We1, be1, We2, be2):
    full = lambda a: pl.BlockSpec(a.shape, lambda i: (0,) * a.ndim)
    return pl.pallas_call(
        _encoder_body,
        grid=(N // NBLK,),
        in_specs=[pl.BlockSpec((NBLK, D_IN), lambda i: (i, 0)),
                  full(We0), full(be0), full(We1), full(be1), full(We2), full(be2)],
        out_specs=pl.BlockSpec((NBLK, D_F), lambda i: (i, 0)),
        out_shape=jax.ShapeDtypeStruct((N, D_F), _F32),
    )(x, We0, be0, We1, be1, We2, be2)


def _sc_gather(h, src, dst):
    mesh = plsc.VectorSubcoreMesh(core_axis_name="c", subcore_axis_name="s")

    @functools.partial(
        pl.kernel,
        out_type=[jax.ShapeDtypeStruct((E, D_F), _F32),
                  jax.ShapeDtypeStruct((E, D_F), _F32)],
        mesh=mesh,
        scratch_types=[pltpu.VMEM((ECH,), jnp.int32),
                       pltpu.VMEM((ECH,), jnp.int32),
                       pltpu.VMEM((ECH, D_F), _F32),
                       pltpu.VMEM((ECH, D_F), _F32),
                       pltpu.SemaphoreType.DMA,
                       pltpu.SemaphoreType.DMA],
    )
    def k(h_hbm, src_hbm, dst_hbm, gs_hbm, gd_hbm, si_v, di_v, sr_v, dr_v, s_sem, d_sem):
        wid = lax.axis_index("s") * NUM_SC + lax.axis_index("c")
        base = wid * EPW

        @pl.loop(0, NCH)
        def _(i):
            off = base + i * ECH
            pltpu.sync_copy(src_hbm.at[pl.ds(off, ECH)], si_v)
            pltpu.sync_copy(dst_hbm.at[pl.ds(off, ECH)], di_v)
            cs = pltpu.async_copy(h_hbm.at[si_v], sr_v, s_sem)
            cd = pltpu.async_copy(h_hbm.at[di_v], dr_v, d_sem)
            cs.wait()
            cd.wait()
            pltpu.sync_copy(sr_v, gs_hbm.at[pl.ds(off, ECH)])
            pltpu.sync_copy(dr_v, gd_hbm.at[pl.ds(off, ECH)])

    return k(h, src, dst)


def _edge_body(gs_ref, gd_ref, ea_ref, w0s, w0d, w0e, b0, w1, b1, w2, b2, m_ref):
    bf = jnp.bfloat16
    t = (_mm(gs_ref[...].astype(bf), w0s[...]) + _mm(gd_ref[...].astype(bf), w0d[...])
         + _mm(ea_ref[...].astype(bf), w0e[...]) + b0[...])
    t = jax.nn.relu(t)
    t = jax.nn.relu(_mm(t.astype(bf), w1[...]) + b1[...])
    m_ref[...] = _mm(t.astype(bf), w2[...]) + b2[...]


def _edge_mlp(gs, gd, ea, Wi0s, Wi0d, Wi0e, bi0, Wi1, bi1, Wi2, bi2):
    full = lambda a: pl.BlockSpec(a.shape, lambda i: (0,) * a.ndim)
    return pl.pallas_call(
        _edge_body,
        grid=(E // EBLK,),
        in_specs=[pl.BlockSpec((EBLK, D_F), lambda i: (i, 0)),
                  pl.BlockSpec((EBLK, D_F), lambda i: (i, 0)),
                  pl.BlockSpec((EBLK, D_E), lambda i: (i, 0)),
                  full(Wi0s), full(Wi0d), full(Wi0e), full(bi0),
                  full(Wi1), full(bi1), full(Wi2), full(bi2)],
        out_specs=pl.BlockSpec((EBLK, D_F), lambda i: (i, 0)),
        out_shape=jax.ShapeDtypeStruct((E, D_F), _F32),
    )(gs, gd, ea, Wi0s, Wi0d, Wi0e, bi0, Wi1, bi1, Wi2, bi2)


def _sc_segsum(m, dst):
    mesh = plsc.VectorSubcoreMesh(core_axis_name="c", subcore_axis_name="s")

    @functools.partial(
        pl.kernel,
        out_type=jax.ShapeDtypeStruct((NUM_SC * NPAD, D_F), _F32),
        mesh=mesh,
        scratch_types=[pltpu.VMEM((ECH, D_F), _F32),
                       pltpu.VMEM((ECH,), jnp.int32),
                       pltpu.VMEM_SHARED((NPAD, D_F), _F32)],
    )
    def agg_k(m_hbm, dst_hbm, agg_hbm, m_v, idx_v, sh_agg):
        cid = lax.axis_index("c")
        sid = lax.axis_index("s")
        wid = sid * NUM_SC + cid

        @pl.loop(0, ECH)
        def _(r):
            @pl.loop(0, D_F, step=16)
            def _(c):
                m_v.at[r, pl.ds(c, 16)][...] = jnp.zeros((16,), _F32)

        @pl.loop(0, ROWS_PER_TILE // ECH)
        def _(j):
            pltpu.sync_copy(m_v, sh_agg.at[pl.ds(sid * ROWS_PER_TILE + j * ECH, ECH)])

        plsc.subcore_barrier()
        base = wid * EPW

        @pl.loop(0, NCH)
        def _(i):
            off = base + i * ECH
            pltpu.sync_copy(m_hbm.at[pl.ds(off, ECH)], m_v)
            pltpu.sync_copy(dst_hbm.at[pl.ds(off, ECH)], idx_v)
            pltpu.sync_copy(m_v, sh_agg.at[idx_v], add=True)

        plsc.subcore_barrier()
        r0 = sid * ROWS_PER_TILE
        pltpu.sync_copy(sh_agg.at[pl.ds(r0, ROWS_PER_TILE)],
                        agg_hbm.at[pl.ds(cid * NPAD + r0, ROWS_PER_TILE)])

    @functools.partial(
        pl.kernel,
        out_type=jax.ShapeDtypeStruct((NUM_SC * NPAD, D_F), _F32),
        mesh=mesh,
        scratch_types=[pltpu.VMEM((ECH, D_F), _F32),
                       pltpu.VMEM((ECH,), jnp.int32),
                       pltpu.VMEM_SHARED((NPAD, D_F), _F32)],
    )
    def cnt_k(dst_hbm, cnt_hbm, ones_v, idx_v, sh_cnt):
        cid = lax.axis_index("c")
        sid = lax.axis_index("s")
        wid = sid * NUM_SC + cid

        @pl.loop(0, ECH)
        def _(r):
            @pl.loop(0, D_F, step=16)
            def _(c):
                ones_v.at[r, pl.ds(c, 16)][...] = jnp.zeros((16,), _F32)

        @pl.loop(0, ROWS_PER_TILE // ECH)
        def _(j):
            pltpu.sync_copy(ones_v, sh_cnt.at[pl.ds(sid * ROWS_PER_TILE + j * ECH, ECH)])

        @pl.loop(0, ECH)
        def _(r):
            @pl.loop(0, D_F, step=16)
            def _(c):
                ones_v.at[r, pl.ds(c, 16)][...] = jnp.full((16,), 1.0, _F32)

        plsc.subcore_barrier()
        base = wid * EPW

        @pl.loop(0, NCH)
        def _(i):
            off = base + i * ECH
            pltpu.sync_copy(dst_hbm.at[pl.ds(off, ECH)], idx_v)
            pltpu.sync_copy(ones_v, sh_cnt.at[idx_v], add=True)

        plsc.subcore_barrier()
        r0 = sid * ROWS_PER_TILE
        pltpu.sync_copy(sh_cnt.at[pl.ds(r0, ROWS_PER_TILE)],
                        cnt_hbm.at[pl.ds(cid * NPAD + r0, ROWS_PER_TILE)])

    return agg_k(m, dst), cnt_k(dst)


def _node_body(h_ref, a0_ref, a1_ref, c0_ref, c1_ref, wn0h, wn0a, bn0, wn1, bn1,
               wn2, bn2, wo0, bo0, wo1, bo1, wo2, bo2, out_ref):
    cnt = c0_ref[...][:, 0:1] + c1_ref[...][:, 0:1]
    agg = (a0_ref[...] + a1_ref[...]) / jnp.maximum(cnt, 1.0)
    t = jax.nn.relu(_mm(h_ref[...], wn0h[...]) + _mm(agg, wn0a[...]) + bn0[...])
    t = jax.nn.relu(_mm(t, wn1[...]) + bn1[...])
    h2 = _mm(t, wn2[...]) + bn2[...]
    t = jax.nn.relu(_mm(h2, wo0[...]) + bo0[...])
    t = jax.nn.relu(_mm(t, wo1[...]) + bo1[...])
    out_ref[...] = _mm(t, wo2[...]) + bo2[...]


def _node_mlp(h, a0, a1, c0, c1, Wn0h, Wn0a, bn0, Wn1, bn1, Wn2, bn2,
              Wo0, bo0, Wo1, bo1, Wo2, bo2):
    full = lambda a: pl.BlockSpec(a.shape, lambda i: (0,) * a.ndim)
    row = lambda d: pl.BlockSpec((NBLK, d), lambda i: (i, 0))
    ws = (Wn0h, Wn0a, bn0, Wn1, bn1, Wn2, bn2, Wo0, bo0, Wo1, bo1, Wo2, bo2)
    return pl.pallas_call(
        _node_body,
        grid=(N // NBLK,),
        in_specs=[row(D_F), row(D_F), row(D_F), row(D_F), row(D_F)]
                 + [full(w) for w in ws],
        out_specs=pl.BlockSpec((NBLK, NCLS), lambda i: (i, 0)),
        out_shape=jax.ShapeDtypeStruct((N, NCLS), _F32),
    )(h, a0, a1, c0, c1, *ws)


def kernel(x, edge_index, edge_attr, We0, be0, We1, be1, We2, be2,
           Wi0, bi0, Wi1, bi1, Wi2, bi2, Wn0, bn0, Wn1, bn1, Wn2, bn2,
           Wo0, bo0, Wo1, bo1, Wo2, bo2):
    src = edge_index[0]
    dst = edge_index[1]
    r1 = lambda b: b.reshape(1, -1)

    h = _encode(x, We0, r1(be0), We1, r1(be1), We2, r1(be2))
    gs, gd = _sc_gather(h, src, dst)
    bf = jnp.bfloat16
    m = _edge_mlp(gs, gd, edge_attr,
                  Wi0[:D_F].astype(bf), Wi0[D_F:2 * D_F].astype(bf),
                  Wi0[2 * D_F:].astype(bf), r1(bi0),
                  Wi1.astype(bf), r1(bi1), Wi2.astype(bf), r1(bi2))
    agg_flat, cnt_flat = _sc_segsum(m, dst)
    aggp = agg_flat.reshape(NUM_SC, NPAD, D_F)[:, :N]
    cntp = cnt_flat.reshape(NUM_SC, NPAD, D_F)[:, :N]
    out = _node_mlp(h, aggp[0], aggp[1], cntp[0], cntp[1],
                    Wn0[:D_F], Wn0[D_F:], r1(bn0), Wn1, r1(bn1), Wn2, r1(bn2),
                    Wo0, r1(bo0), Wo1, r1(bo1), Wo2, r1(bo2))
    return out


# speedup vs baseline: 2.2965x; 1.0961x over previous
import functools

import jax
import jax.numpy as jnp
from jax import lax
from jax.experimental import pallas as pl
from jax.experimental.pallas import tpu as pltpu
from jax.experimental.pallas import tpu_sc as plsc

N = 10000
E = 320000
D_IN = 128
D_F = 128
D_H = 256
D_E = 16
NCLS = 10

NUM_SC = 2
NUM_TILES = 16
NW = NUM_SC * NUM_TILES
EPW = E // NW
ECH = 80
NCH = EPW // ECH
NPAD = 10240
ROWS_PER_TILE = NPAD // NUM_TILES

NBLK = 1000
EBLK = 512

_F32 = jnp.float32


def _mm(a, b):
    return jnp.dot(a, b, preferred_element_type=jnp.float32)


def _encoder_body(x_ref, w0, b0, w1, b1, w2, b2, h_ref):
    t = jax.nn.relu(_mm(x_ref[...], w0[...]) + b0[...])
    t = jax.nn.relu(_mm(t, w1[...]) + b1[...])
    h_ref[...] = _mm(t, w2[...]) + b2[...]


def _encode(x, We0, be0, We1, be1, We2, be2):
    full = lambda a: pl.BlockSpec(a.shape, lambda i: (0,) * a.ndim)
    return pl.pallas_call(
        _encoder_body,
        grid=(N // NBLK,),
        in_specs=[pl.BlockSpec((NBLK, D_IN), lambda i: (i, 0)),
                  full(We0), full(be0), full(We1), full(be1), full(We2), full(be2)],
        out_specs=pl.BlockSpec((NBLK, D_F), lambda i: (i, 0)),
        out_shape=jax.ShapeDtypeStruct((N, D_F), _F32),
    )(x, We0, be0, We1, be1, We2, be2)


SCH = 400
NSUB = SCH // ECH
NSCH = EPW // SCH


def _sc_gather(h, src, dst):
    mesh = plsc.VectorSubcoreMesh(core_axis_name="c", subcore_axis_name="s")

    @functools.partial(
        pl.kernel,
        out_type=[jax.ShapeDtypeStruct((E, D_F), _F32),
                  jax.ShapeDtypeStruct((E, D_F), _F32)],
        mesh=mesh,
        scratch_types=[pltpu.VMEM((SCH,), jnp.int32),
                       pltpu.VMEM((SCH,), jnp.int32),
                       pltpu.VMEM((SCH, D_F), _F32),
                       pltpu.VMEM((SCH, D_F), _F32),
                       pltpu.SemaphoreType.DMA,
                       pltpu.SemaphoreType.DMA,
                       pltpu.SemaphoreType.DMA],
    )
    def k(h_hbm, src_hbm, dst_hbm, gs_hbm, gd_hbm, si_v, di_v, sr_v, dr_v,
          i_sem, g_sem, w_sem):
        wid = lax.axis_index("s") * NUM_SC + lax.axis_index("c")
        base = wid * EPW

        @pl.loop(0, NSCH)
        def _(s):
            off = base + s * SCH
            ci = pltpu.async_copy(src_hbm.at[pl.ds(off, SCH)], si_v, i_sem)
            cd = pltpu.async_copy(dst_hbm.at[pl.ds(off, SCH)], di_v, i_sem)
            ci.wait()
            cd.wait()
            gathers = []
            for j in range(NSUB):
                sl = pl.ds(j * ECH, ECH)
                gathers.append(pltpu.async_copy(
                    h_hbm.at[si_v.at[sl]], sr_v.at[sl], g_sem))
                gathers.append(pltpu.async_copy(
                    h_hbm.at[di_v.at[sl]], dr_v.at[sl], g_sem))
            for g in gathers:
                g.wait()
            ws = pltpu.async_copy(sr_v, gs_hbm.at[pl.ds(off, SCH)], w_sem)
            wd = pltpu.async_copy(dr_v, gd_hbm.at[pl.ds(off, SCH)], w_sem)
            ws.wait()
            wd.wait()

    return k(h, src, dst)


def _edge_body(gs_ref, gd_ref, ea_ref, w0s, w0d, w0e, b0, w1, b1, w2, b2, m_ref):
    bf = jnp.bfloat16
    t = (_mm(gs_ref[...].astype(bf), w0s[...]) + _mm(gd_ref[...].astype(bf), w0d[...])
         + _mm(ea_ref[...].astype(bf), w0e[...]) + b0[...])
    t = jax.nn.relu(t)
    t = jax.nn.relu(_mm(t.astype(bf), w1[...]) + b1[...])
    m_ref[...] = _mm(t.astype(bf), w2[...]) + b2[...]


def _edge_mlp(gs, gd, ea, Wi0s, Wi0d, Wi0e, bi0, Wi1, bi1, Wi2, bi2):
    full = lambda a: pl.BlockSpec(a.shape, lambda i: (0,) * a.ndim)
    return pl.pallas_call(
        _edge_body,
        grid=(E // EBLK,),
        in_specs=[pl.BlockSpec((EBLK, D_F), lambda i: (i, 0)),
                  pl.BlockSpec((EBLK, D_F), lambda i: (i, 0)),
                  pl.BlockSpec((EBLK, D_E), lambda i: (i, 0)),
                  full(Wi0s), full(Wi0d), full(Wi0e), full(bi0),
                  full(Wi1), full(bi1), full(Wi2), full(bi2)],
        out_specs=pl.BlockSpec((EBLK, D_F), lambda i: (i, 0)),
        out_shape=jax.ShapeDtypeStruct((E, D_F), _F32),
    )(gs, gd, ea, Wi0s, Wi0d, Wi0e, bi0, Wi1, bi1, Wi2, bi2)


def _sc_segsum(m, dst):
    mesh = plsc.VectorSubcoreMesh(core_axis_name="c", subcore_axis_name="s")

    @functools.partial(
        pl.kernel,
        out_type=jax.ShapeDtypeStruct((NUM_SC * NPAD, D_F), _F32),
        mesh=mesh,
        scratch_types=[pltpu.VMEM((ECH, D_F), _F32),
                       pltpu.VMEM((ECH,), jnp.int32),
                       pltpu.VMEM_SHARED((NPAD, D_F), _F32)],
    )
    def agg_k(m_hbm, dst_hbm, agg_hbm, m_v, idx_v, sh_agg):
        cid = lax.axis_index("c")
        sid = lax.axis_index("s")
        wid = sid * NUM_SC + cid

        @pl.loop(0, ECH)
        def _(r):
            @pl.loop(0, D_F, step=16)
            def _(c):
                m_v.at[r, pl.ds(c, 16)][...] = jnp.zeros((16,), _F32)

        @pl.loop(0, ROWS_PER_TILE // ECH)
        def _(j):
            pltpu.sync_copy(m_v, sh_agg.at[pl.ds(sid * ROWS_PER_TILE + j * ECH, ECH)])

        plsc.subcore_barrier()
        base = wid * EPW

        @pl.loop(0, NCH)
        def _(i):
            off = base + i * ECH
            pltpu.sync_copy(m_hbm.at[pl.ds(off, ECH)], m_v)
            pltpu.sync_copy(dst_hbm.at[pl.ds(off, ECH)], idx_v)
            pltpu.sync_copy(m_v, sh_agg.at[idx_v], add=True)

        plsc.subcore_barrier()
        r0 = sid * ROWS_PER_TILE
        pltpu.sync_copy(sh_agg.at[pl.ds(r0, ROWS_PER_TILE)],
                        agg_hbm.at[pl.ds(cid * NPAD + r0, ROWS_PER_TILE)])

    @functools.partial(
        pl.kernel,
        out_type=jax.ShapeDtypeStruct((NUM_SC * NPAD, D_F), _F32),
        mesh=mesh,
        scratch_types=[pltpu.VMEM((ECH, D_F), _F32),
                       pltpu.VMEM((ECH,), jnp.int32),
                       pltpu.VMEM_SHARED((NPAD, D_F), _F32)],
    )
    def cnt_k(dst_hbm, cnt_hbm, ones_v, idx_v, sh_cnt):
        cid = lax.axis_index("c")
        sid = lax.axis_index("s")
        wid = sid * NUM_SC + cid

        @pl.loop(0, ECH)
        def _(r):
            @pl.loop(0, D_F, step=16)
            def _(c):
                ones_v.at[r, pl.ds(c, 16)][...] = jnp.zeros((16,), _F32)

        @pl.loop(0, ROWS_PER_TILE // ECH)
        def _(j):
            pltpu.sync_copy(ones_v, sh_cnt.at[pl.ds(sid * ROWS_PER_TILE + j * ECH, ECH)])

        @pl.loop(0, ECH)
        def _(r):
            @pl.loop(0, D_F, step=16)
            def _(c):
                ones_v.at[r, pl.ds(c, 16)][...] = jnp.full((16,), 1.0, _F32)

        plsc.subcore_barrier()
        base = wid * EPW

        @pl.loop(0, NCH)
        def _(i):
            off = base + i * ECH
            pltpu.sync_copy(dst_hbm.at[pl.ds(off, ECH)], idx_v)
            pltpu.sync_copy(ones_v, sh_cnt.at[idx_v], add=True)

        plsc.subcore_barrier()
        r0 = sid * ROWS_PER_TILE
        pltpu.sync_copy(sh_cnt.at[pl.ds(r0, ROWS_PER_TILE)],
                        cnt_hbm.at[pl.ds(cid * NPAD + r0, ROWS_PER_TILE)])

    return agg_k(m, dst), cnt_k(dst)


def _node_body(h_ref, a0_ref, a1_ref, c0_ref, c1_ref, wn0h, wn0a, bn0, wn1, bn1,
               wn2, bn2, wo0, bo0, wo1, bo1, wo2, bo2, out_ref):
    cnt = c0_ref[...][:, 0:1] + c1_ref[...][:, 0:1]
    agg = (a0_ref[...] + a1_ref[...]) / jnp.maximum(cnt, 1.0)
    t = jax.nn.relu(_mm(h_ref[...], wn0h[...]) + _mm(agg, wn0a[...]) + bn0[...])
    t = jax.nn.relu(_mm(t, wn1[...]) + bn1[...])
    h2 = _mm(t, wn2[...]) + bn2[...]
    t = jax.nn.relu(_mm(h2, wo0[...]) + bo0[...])
    t = jax.nn.relu(_mm(t, wo1[...]) + bo1[...])
    out_ref[...] = _mm(t, wo2[...]) + bo2[...]


def _node_mlp(h, a0, a1, c0, c1, Wn0h, Wn0a, bn0, Wn1, bn1, Wn2, bn2,
              Wo0, bo0, Wo1, bo1, Wo2, bo2):
    full = lambda a: pl.BlockSpec(a.shape, lambda i: (0,) * a.ndim)
    row = lambda d: pl.BlockSpec((NBLK, d), lambda i: (i, 0))
    ws = (Wn0h, Wn0a, bn0, Wn1, bn1, Wn2, bn2, Wo0, bo0, Wo1, bo1, Wo2, bo2)
    return pl.pallas_call(
        _node_body,
        grid=(N // NBLK,),
        in_specs=[row(D_F), row(D_F), row(D_F), row(D_F), row(D_F)]
                 + [full(w) for w in ws],
        out_specs=pl.BlockSpec((NBLK, NCLS), lambda i: (i, 0)),
        out_shape=jax.ShapeDtypeStruct((N, NCLS), _F32),
    )(h, a0, a1, c0, c1, *ws)


def kernel(x, edge_index, edge_attr, We0, be0, We1, be1, We2, be2,
           Wi0, bi0, Wi1, bi1, Wi2, bi2, Wn0, bn0, Wn1, bn1, Wn2, bn2,
           Wo0, bo0, Wo1, bo1, Wo2, bo2):
    src = edge_index[0]
    dst = edge_index[1]
    r1 = lambda b: b.reshape(1, -1)

    h = _encode(x, We0, r1(be0), We1, r1(be1), We2, r1(be2))
    gs, gd = _sc_gather(h, src, dst)
    bf = jnp.bfloat16
    m = _edge_mlp(gs, gd, edge_attr,
                  Wi0[:D_F].astype(bf), Wi0[D_F:2 * D_F].astype(bf),
                  Wi0[2 * D_F:].astype(bf), r1(bi0),
                  Wi1.astype(bf), r1(bi1), Wi2.astype(bf), r1(bi2))
    agg_flat, cnt_flat = _sc_segsum(m, dst)
    aggp = agg_flat.reshape(NUM_SC, NPAD, D_F)[:, :N]
    cntp = cnt_flat.reshape(NUM_SC, NPAD, D_F)[:, :N]
    out = _node_mlp(h, aggp[0], aggp[1], cntp[0], cntp[1],
                    Wn0[:D_F], Wn0[D_F:], r1(bn0), Wn1, r1(bn1), Wn2, r1(bn2),
                    Wo0, r1(bo0), Wo1, r1(bo1), Wo2, r1(bo2))
    return out


# speedup vs baseline: 2.5300x; 1.1017x over previous
import functools

import jax
import jax.numpy as jnp
from jax import lax
from jax.experimental import pallas as pl
from jax.experimental.pallas import tpu as pltpu
from jax.experimental.pallas import tpu_sc as plsc

N = 10000
E = 320000
D_IN = 128
D_F = 128
D_H = 256
D_E = 16
NCLS = 10

NUM_SC = 2
NUM_TILES = 16
NW = NUM_SC * NUM_TILES
EPW = E // NW
ECH = 80
NCH = EPW // ECH
NPAD = 10240
ROWS_PER_TILE = NPAD // NUM_TILES

NBLK = 1000
EBLK = 512

_F32 = jnp.float32


def _mm(a, b):
    return jnp.dot(a, b, preferred_element_type=jnp.float32)


def _encoder_body(x_ref, w0, b0, w1, b1, w2, b2, h_ref):
    t = jax.nn.relu(_mm(x_ref[...], w0[...]) + b0[...])
    t = jax.nn.relu(_mm(t, w1[...]) + b1[...])
    h_ref[...] = _mm(t, w2[...]) + b2[...]


def _encode(x, We0, be0, We1, be1, We2, be2):
    full = lambda a: pl.BlockSpec(a.shape, lambda i: (0,) * a.ndim)
    return pl.pallas_call(
        _encoder_body,
        grid=(N // NBLK,),
        in_specs=[pl.BlockSpec((NBLK, D_IN), lambda i: (i, 0)),
                  full(We0), full(be0), full(We1), full(be1), full(We2), full(be2)],
        out_specs=pl.BlockSpec((NBLK, D_F), lambda i: (i, 0)),
        out_shape=jax.ShapeDtypeStruct((N, D_F), _F32),
    )(x, We0, be0, We1, be1, We2, be2)


SCH = 200
GSUB = 5
GCH = SCH // GSUB
NSCH = EPW // SCH


def _sc_gather(h, src, dst):
    mesh = plsc.VectorSubcoreMesh(core_axis_name="c", subcore_axis_name="s")

    @functools.partial(
        pl.kernel,
        out_type=[jax.ShapeDtypeStruct((E, D_F), _F32),
                  jax.ShapeDtypeStruct((E, D_F), _F32)],
        mesh=mesh,
        scratch_types=[pltpu.VMEM((SCH,), jnp.int32),
                       pltpu.VMEM((SCH,), jnp.int32),
                       pltpu.VMEM((SCH, D_F), _F32),
                       pltpu.VMEM((SCH, D_F), _F32),
                       pltpu.VMEM((SCH,), jnp.int32),
                       pltpu.VMEM((SCH,), jnp.int32),
                       pltpu.VMEM((SCH, D_F), _F32),
                       pltpu.VMEM((SCH, D_F), _F32),
                       pltpu.SemaphoreType.DMA,
                       pltpu.SemaphoreType.DMA,
                       pltpu.SemaphoreType.DMA,
                       pltpu.SemaphoreType.DMA,
                       pltpu.SemaphoreType.DMA],
    )
    def k(h_hbm, src_hbm, dst_hbm, gs_hbm, gd_hbm,
          siA, diA, srA, drA, siB, diB, srB, drB,
          lA_sem, lB_sem, g_sem, wA_sem, wB_sem):
        wid = lax.axis_index("s") * NUM_SC + lax.axis_index("c")
        base = wid * EPW

        def issue_loads(s, si, di, lsem):
            off = base + s * SCH
            pltpu.async_copy(src_hbm.at[pl.ds(off, SCH)], si, lsem)
            pltpu.async_copy(dst_hbm.at[pl.ds(off, SCH)], di, lsem)

        def wait_loads(si, di, lsem):
            pltpu.make_async_copy(src_hbm.at[pl.ds(0, SCH)], si, lsem).wait()
            pltpu.make_async_copy(dst_hbm.at[pl.ds(0, SCH)], di, lsem).wait()

        def wait_writes(sr, dr, wsem):
            pltpu.make_async_copy(sr, gs_hbm.at[pl.ds(0, SCH)], wsem).wait()
            pltpu.make_async_copy(dr, gd_hbm.at[pl.ds(0, SCH)], wsem).wait()

        def step(s, si, di, sr, dr, lsem, wsem, nsi, ndi, nlsem):
            wait_loads(si, di, lsem)

            @pl.when(s > 1)
            def _():
                wait_writes(sr, dr, wsem)

            gathers = []
            for j in range(GSUB):
                sl = pl.ds(j * GCH, GCH)
                gathers.append(pltpu.async_copy(h_hbm.at[si.at[sl]], sr.at[sl], g_sem))
                gathers.append(pltpu.async_copy(h_hbm.at[di.at[sl]], dr.at[sl], g_sem))

            @pl.when(s < NSCH - 1)
            def _():
                issue_loads(s + 1, nsi, ndi, nlsem)

            for g in gathers:
                g.wait()
            off = base + s * SCH
            pltpu.async_copy(sr, gs_hbm.at[pl.ds(off, SCH)], wsem)
            pltpu.async_copy(dr, gd_hbm.at[pl.ds(off, SCH)], wsem)

        issue_loads(0, siA, diA, lA_sem)

        @pl.loop(0, NSCH)
        def _(s):
            @pl.when(s % 2 == 0)
            def _():
                step(s, siA, diA, srA, drA, lA_sem, wA_sem, siB, diB, lB_sem)

            @pl.when(s % 2 == 1)
            def _():
                step(s, siB, diB, srB, drB, lB_sem, wB_sem, siA, diA, lA_sem)

        wait_writes(srA, drA, wA_sem)
        wait_writes(srB, drB, wB_sem)

    return k(h, src, dst)


def _edge_body(gs_ref, gd_ref, ea_ref, w0s, w0d, w0e, b0, w1, b1, w2, b2, m_ref):
    bf = jnp.bfloat16
    t = (_mm(gs_ref[...].astype(bf), w0s[...]) + _mm(gd_ref[...].astype(bf), w0d[...])
         + _mm(ea_ref[...].astype(bf), w0e[...]) + b0[...])
    t = jax.nn.relu(t)
    t = jax.nn.relu(_mm(t.astype(bf), w1[...]) + b1[...])
    m_ref[...] = _mm(t.astype(bf), w2[...]) + b2[...]


def _edge_mlp(gs, gd, ea, Wi0s, Wi0d, Wi0e, bi0, Wi1, bi1, Wi2, bi2):
    full = lambda a: pl.BlockSpec(a.shape, lambda i: (0,) * a.ndim)
    return pl.pallas_call(
        _edge_body,
        grid=(E // EBLK,),
        in_specs=[pl.BlockSpec((EBLK, D_F), lambda i: (i, 0)),
                  pl.BlockSpec((EBLK, D_F), lambda i: (i, 0)),
                  pl.BlockSpec((EBLK, D_E), lambda i: (i, 0)),
                  full(Wi0s), full(Wi0d), full(Wi0e), full(bi0),
                  full(Wi1), full(bi1), full(Wi2), full(bi2)],
        out_specs=pl.BlockSpec((EBLK, D_F), lambda i: (i, 0)),
        out_shape=jax.ShapeDtypeStruct((E, D_F), _F32),
    )(gs, gd, ea, Wi0s, Wi0d, Wi0e, bi0, Wi1, bi1, Wi2, bi2)


def _sc_segsum(m, dst):
    mesh = plsc.VectorSubcoreMesh(core_axis_name="c", subcore_axis_name="s")

    @functools.partial(
        pl.kernel,
        out_type=jax.ShapeDtypeStruct((NUM_SC * NPAD, D_F), _F32),
        mesh=mesh,
        scratch_types=[pltpu.VMEM((ECH, D_F), _F32),
                       pltpu.VMEM((ECH, D_F), _F32),
                       pltpu.VMEM((ECH,), jnp.int32),
                       pltpu.VMEM((ECH,), jnp.int32),
                       pltpu.VMEM_SHARED((NPAD, D_F), _F32),
                       pltpu.SemaphoreType.DMA,
                       pltpu.SemaphoreType.DMA],
    )
    def agg_k(m_hbm, dst_hbm, agg_hbm, mA, mB, iA, iB, sh_agg, lA_sem, lB_sem):
        cid = lax.axis_index("c")
        sid = lax.axis_index("s")
        wid = sid * NUM_SC + cid
        base = wid * EPW

        @pl.loop(0, ECH)
        def _(r):
            @pl.loop(0, D_F, step=16)
            def _(c):
                mA.at[r, pl.ds(c, 16)][...] = jnp.zeros((16,), _F32)

        @pl.loop(0, ROWS_PER_TILE // ECH)
        def _(j):
            pltpu.sync_copy(mA, sh_agg.at[pl.ds(sid * ROWS_PER_TILE + j * ECH, ECH)])

        plsc.subcore_barrier()

        def issue_loads(s, mbuf, ibuf, lsem):
            off = base + s * ECH
            pltpu.async_copy(m_hbm.at[pl.ds(off, ECH)], mbuf, lsem)
            pltpu.async_copy(dst_hbm.at[pl.ds(off, ECH)], ibuf, lsem)

        def step(s, mc, ic, lc, mn, inx, ln):
            pltpu.make_async_copy(m_hbm.at[pl.ds(0, ECH)], mc, lc).wait()
            pltpu.make_async_copy(dst_hbm.at[pl.ds(0, ECH)], ic, lc).wait()

            @pl.when(s < NCH - 1)
            def _():
                issue_loads(s + 1, mn, inx, ln)

            pltpu.sync_copy(mc, sh_agg.at[ic], add=True)

        issue_loads(0, mA, iA, lA_sem)

        @pl.loop(0, NCH)
        def _(s):
            @pl.when(s % 2 == 0)
            def _():
                step(s, mA, iA, lA_sem, mB, iB, lB_sem)

            @pl.when(s % 2 == 1)
            def _():
                step(s, mB, iB, lB_sem, mA, iA, lA_sem)

        plsc.subcore_barrier()
        r0 = sid * ROWS_PER_TILE
        pltpu.sync_copy(sh_agg.at[pl.ds(r0, ROWS_PER_TILE)],
                        agg_hbm.at[pl.ds(cid * NPAD + r0, ROWS_PER_TILE)])

    @functools.partial(
        pl.kernel,
        out_type=jax.ShapeDtypeStruct((NUM_SC * NPAD, D_F), _F32),
        mesh=mesh,
        scratch_types=[pltpu.VMEM((ECH, D_F), _F32),
                       pltpu.VMEM((ECH,), jnp.int32),
                       pltpu.VMEM((ECH,), jnp.int32),
                       pltpu.VMEM_SHARED((NPAD, D_F), _F32),
                       pltpu.SemaphoreType.DMA,
                       pltpu.SemaphoreType.DMA],
    )
    def cnt_k(dst_hbm, cnt_hbm, ones_v, iA, iB, sh_cnt, lA_sem, lB_sem):
        cid = lax.axis_index("c")
        sid = lax.axis_index("s")
        wid = sid * NUM_SC + cid
        base = wid * EPW

        @pl.loop(0, ECH)
        def _(r):
            @pl.loop(0, D_F, step=16)
            def _(c):
                ones_v.at[r, pl.ds(c, 16)][...] = jnp.zeros((16,), _F32)

        @pl.loop(0, ROWS_PER_TILE // ECH)
        def _(j):
            pltpu.sync_copy(ones_v, sh_cnt.at[pl.ds(sid * ROWS_PER_TILE + j * ECH, ECH)])

        @pl.loop(0, ECH)
        def _(r):
            @pl.loop(0, D_F, step=16)
            def _(c):
                ones_v.at[r, pl.ds(c, 16)][...] = jnp.full((16,), 1.0, _F32)

        plsc.subcore_barrier()

        def step(s, ic, lc, inx, ln):
            pltpu.make_async_copy(dst_hbm.at[pl.ds(0, ECH)], ic, lc).wait()

            @pl.when(s < NCH - 1)
            def _():
                pltpu.async_copy(dst_hbm.at[pl.ds(base + (s + 1) * ECH, ECH)], inx, ln)

            pltpu.sync_copy(ones_v, sh_cnt.at[ic], add=True)

        pltpu.async_copy(dst_hbm.at[pl.ds(base, ECH)], iA, lA_sem)

        @pl.loop(0, NCH)
        def _(s):
            @pl.when(s % 2 == 0)
            def _():
                step(s, iA, lA_sem, iB, lB_sem)

            @pl.when(s % 2 == 1)
            def _():
                step(s, iB, lB_sem, iA, lA_sem)

        plsc.subcore_barrier()
        r0 = sid * ROWS_PER_TILE
        pltpu.sync_copy(sh_cnt.at[pl.ds(r0, ROWS_PER_TILE)],
                        cnt_hbm.at[pl.ds(cid * NPAD + r0, ROWS_PER_TILE)])

    return agg_k(m, dst), cnt_k(dst)


def _node_body(h_ref, a0_ref, a1_ref, c0_ref, c1_ref, wn0h, wn0a, bn0, wn1, bn1,
               wn2, bn2, wo0, bo0, wo1, bo1, wo2, bo2, out_ref):
    cnt = c0_ref[...][:, 0:1] + c1_ref[...][:, 0:1]
    agg = (a0_ref[...] + a1_ref[...]) / jnp.maximum(cnt, 1.0)
    t = jax.nn.relu(_mm(h_ref[...], wn0h[...]) + _mm(agg, wn0a[...]) + bn0[...])
    t = jax.nn.relu(_mm(t, wn1[...]) + bn1[...])
    h2 = _mm(t, wn2[...]) + bn2[...]
    t = jax.nn.relu(_mm(h2, wo0[...]) + bo0[...])
    t = jax.nn.relu(_mm(t, wo1[...]) + bo1[...])
    out_ref[...] = _mm(t, wo2[...]) + bo2[...]


def _node_mlp(h, a0, a1, c0, c1, Wn0h, Wn0a, bn0, Wn1, bn1, Wn2, bn2,
              Wo0, bo0, Wo1, bo1, Wo2, bo2):
    full = lambda a: pl.BlockSpec(a.shape, lambda i: (0,) * a.ndim)
    row = lambda d: pl.BlockSpec((NBLK, d), lambda i: (i, 0))
    ws = (Wn0h, Wn0a, bn0, Wn1, bn1, Wn2, bn2, Wo0, bo0, Wo1, bo1, Wo2, bo2)
    return pl.pallas_call(
        _node_body,
        grid=(N // NBLK,),
        in_specs=[row(D_F), row(D_F), row(D_F), row(D_F), row(D_F)]
                 + [full(w) for w in ws],
        out_specs=pl.BlockSpec((NBLK, NCLS), lambda i: (i, 0)),
        out_shape=jax.ShapeDtypeStruct((N, NCLS), _F32),
    )(h, a0, a1, c0, c1, *ws)


def kernel(x, edge_index, edge_attr, We0, be0, We1, be1, We2, be2,
           Wi0, bi0, Wi1, bi1, Wi2, bi2, Wn0, bn0, Wn1, bn1, Wn2, bn2,
           Wo0, bo0, Wo1, bo1, Wo2, bo2):
    src = edge_index[0]
    dst = edge_index[1]
    r1 = lambda b: b.reshape(1, -1)

    h = _encode(x, We0, r1(be0), We1, r1(be1), We2, r1(be2))
    gs, gd = _sc_gather(h, src, dst)
    bf = jnp.bfloat16
    m = _edge_mlp(gs, gd, edge_attr,
                  Wi0[:D_F].astype(bf), Wi0[D_F:2 * D_F].astype(bf),
                  Wi0[2 * D_F:].astype(bf), r1(bi0),
                  Wi1.astype(bf), r1(bi1), Wi2.astype(bf), r1(bi2))
    agg_flat, cnt_flat = _sc_segsum(m, dst)
    aggp = agg_flat.reshape(NUM_SC, NPAD, D_F)[:, :N]
    cntp = cnt_flat.reshape(NUM_SC, NPAD, D_F)[:, :N]
    out = _node_mlp(h, aggp[0], aggp[1], cntp[0], cntp[1],
                    Wn0[:D_F], Wn0[D_F:], r1(bn0), Wn1, r1(bn1), Wn2, r1(bn2),
                    Wo0, r1(bo0), Wo1, r1(bo1), Wo2, r1(bo2))
    return out


# speedup vs baseline: 3.1097x; 1.2291x over previous
import functools

import jax
import jax.numpy as jnp
from jax import lax
from jax.experimental import pallas as pl
from jax.experimental.pallas import tpu as pltpu
from jax.experimental.pallas import tpu_sc as plsc

N = 10000
E = 320000
D_IN = 128
D_F = 128
D_H = 256
D_E = 16
NCLS = 10

NUM_SC = 2
NUM_TILES = 16
NW = NUM_SC * NUM_TILES
EPW = E // NW
ECH = 80
NCH = EPW // ECH
NPAD = 10240
ROWS_PER_TILE = NPAD // NUM_TILES

NBLK = 1000
EBLK = 1280

_F32 = jnp.float32


def _mm(a, b):
    return jnp.dot(a, b, preferred_element_type=jnp.float32)


def _encoder_body(x_ref, w0, b0, w1, b1, w2, b2, h_ref):
    t = jax.nn.relu(_mm(x_ref[...], w0[...]) + b0[...])
    t = jax.nn.relu(_mm(t, w1[...]) + b1[...])
    h_ref[...] = _mm(t, w2[...]) + b2[...]


def _encode(x, We0, be0, We1, be1, We2, be2):
    full = lambda a: pl.BlockSpec(a.shape, lambda i: (0,) * a.ndim)
    return pl.pallas_call(
        _encoder_body,
        grid=(N // NBLK,),
        in_specs=[pl.BlockSpec((NBLK, D_IN), lambda i: (i, 0)),
                  full(We0), full(be0), full(We1), full(be1), full(We2), full(be2)],
        out_specs=pl.BlockSpec((NBLK, D_F), lambda i: (i, 0)),
        out_shape=jax.ShapeDtypeStruct((N, D_F), _F32),
    )(x, We0, be0, We1, be1, We2, be2)


SCH = 200
GSUB = 5
GCH = SCH // GSUB
NSCH = EPW // SCH


def _sc_gather(h, src, dst):
    mesh = plsc.VectorSubcoreMesh(core_axis_name="c", subcore_axis_name="s")

    @functools.partial(
        pl.kernel,
        out_type=[jax.ShapeDtypeStruct((E, D_F), _F32),
                  jax.ShapeDtypeStruct((E, D_F), _F32)],
        mesh=mesh,
        scratch_types=[pltpu.VMEM((SCH,), jnp.int32),
                       pltpu.VMEM((SCH,), jnp.int32),
                       pltpu.VMEM((SCH, D_F), _F32),
                       pltpu.VMEM((SCH, D_F), _F32),
                       pltpu.VMEM((SCH,), jnp.int32),
                       pltpu.VMEM((SCH,), jnp.int32),
                       pltpu.VMEM((SCH, D_F), _F32),
                       pltpu.VMEM((SCH, D_F), _F32),
                       pltpu.SemaphoreType.DMA,
                       pltpu.SemaphoreType.DMA,
                       pltpu.SemaphoreType.DMA,
                       pltpu.SemaphoreType.DMA,
                       pltpu.SemaphoreType.DMA],
    )
    def k(h_hbm, src_hbm, dst_hbm, gs_hbm, gd_hbm,
          siA, diA, srA, drA, siB, diB, srB, drB,
          lA_sem, lB_sem, g_sem, wA_sem, wB_sem):
        wid = lax.axis_index("s") * NUM_SC + lax.axis_index("c")
        base = wid * EPW

        def issue_loads(s, si, di, lsem):
            off = base + s * SCH
            pltpu.async_copy(src_hbm.at[pl.ds(off, SCH)], si, lsem)
            pltpu.async_copy(dst_hbm.at[pl.ds(off, SCH)], di, lsem)

        def wait_loads(si, di, lsem):
            pltpu.make_async_copy(src_hbm.at[pl.ds(0, SCH)], si, lsem).wait()
            pltpu.make_async_copy(dst_hbm.at[pl.ds(0, SCH)], di, lsem).wait()

        def wait_writes(sr, dr, wsem):
            pltpu.make_async_copy(sr, gs_hbm.at[pl.ds(0, SCH)], wsem).wait()
            pltpu.make_async_copy(dr, gd_hbm.at[pl.ds(0, SCH)], wsem).wait()

        def step(s, si, di, sr, dr, lsem, wsem, nsi, ndi, nlsem):
            wait_loads(si, di, lsem)

            @pl.when(s > 1)
            def _():
                wait_writes(sr, dr, wsem)

            gathers = []
            for j in range(GSUB):
                sl = pl.ds(j * GCH, GCH)
                gathers.append(pltpu.async_copy(h_hbm.at[si.at[sl]], sr.at[sl], g_sem))
                gathers.append(pltpu.async_copy(h_hbm.at[di.at[sl]], dr.at[sl], g_sem))

            @pl.when(s < NSCH - 1)
            def _():
                issue_loads(s + 1, nsi, ndi, nlsem)

            for g in gathers:
                g.wait()
            off = base + s * SCH
            pltpu.async_copy(sr, gs_hbm.at[pl.ds(off, SCH)], wsem)
            pltpu.async_copy(dr, gd_hbm.at[pl.ds(off, SCH)], wsem)

        issue_loads(0, siA, diA, lA_sem)

        @pl.loop(0, NSCH)
        def _(s):
            @pl.when(s % 2 == 0)
            def _():
                step(s, siA, diA, srA, drA, lA_sem, wA_sem, siB, diB, lB_sem)

            @pl.when(s % 2 == 1)
            def _():
                step(s, siB, diB, srB, drB, lB_sem, wB_sem, siA, diA, lA_sem)

        wait_writes(srA, drA, wA_sem)
        wait_writes(srB, drB, wB_sem)

    return k(h, src, dst)


def _edge_body(gs_ref, gd_ref, ea_ref, w0s, w0d, w0e, b0, w1, b1, w2, b2, m_ref):
    bf = jnp.bfloat16
    t = (_mm(gs_ref[...].astype(bf), w0s[...]) + _mm(gd_ref[...].astype(bf), w0d[...])
         + _mm(ea_ref[...].astype(bf), w0e[...]) + b0[...])
    t = jax.nn.relu(t)
    t = jax.nn.relu(_mm(t.astype(bf), w1[...]) + b1[...])
    m_ref[...] = _mm(t.astype(bf), w2[...]) + b2[...]


def _edge_mlp(gs, gd, ea, Wi0s, Wi0d, Wi0e, bi0, Wi1, bi1, Wi2, bi2):
    full = lambda a: pl.BlockSpec(a.shape, lambda i: (0,) * a.ndim)
    return pl.pallas_call(
        _edge_body,
        grid=(E // EBLK,),
        in_specs=[pl.BlockSpec((EBLK, D_F), lambda i: (i, 0)),
                  pl.BlockSpec((EBLK, D_F), lambda i: (i, 0)),
                  pl.BlockSpec((EBLK, D_E), lambda i: (i, 0)),
                  full(Wi0s), full(Wi0d), full(Wi0e), full(bi0),
                  full(Wi1), full(bi1), full(Wi2), full(bi2)],
        out_specs=pl.BlockSpec((EBLK, D_F), lambda i: (i, 0)),
        out_shape=jax.ShapeDtypeStruct((E, D_F), _F32),
    )(gs, gd, ea, Wi0s, Wi0d, Wi0e, bi0, Wi1, bi1, Wi2, bi2)


def _sc_agg(m, dst):
    mesh = plsc.VectorSubcoreMesh(core_axis_name="c", subcore_axis_name="s")

    @functools.partial(
        pl.kernel,
        out_type=jax.ShapeDtypeStruct((NUM_SC * NPAD, D_F), _F32),
        mesh=mesh,
        scratch_types=[pltpu.VMEM((ECH, D_F), _F32),
                       pltpu.VMEM((ECH, D_F), _F32),
                       pltpu.VMEM((ECH,), jnp.int32),
                       pltpu.VMEM((ECH,), jnp.int32),
                       pltpu.VMEM_SHARED((NPAD, D_F), _F32),
                       pltpu.SemaphoreType.DMA,
                       pltpu.SemaphoreType.DMA],
    )
    def agg_k(m_hbm, dst_hbm, agg_hbm, mA, mB, iA, iB, sh_agg, lA_sem, lB_sem):
        cid = lax.axis_index("c")
        sid = lax.axis_index("s")
        wid = sid * NUM_SC + cid
        base = wid * EPW

        @pl.loop(0, ECH)
        def _(r):
            @pl.loop(0, D_F, step=16)
            def _(c):
                mA.at[r, pl.ds(c, 16)][...] = jnp.zeros((16,), _F32)

        @pl.loop(0, ROWS_PER_TILE // ECH)
        def _(j):
            pltpu.sync_copy(mA, sh_agg.at[pl.ds(sid * ROWS_PER_TILE + j * ECH, ECH)])

        plsc.subcore_barrier()

        def issue_loads(s, mbuf, ibuf, lsem):
            off = base + s * ECH
            pltpu.async_copy(m_hbm.at[pl.ds(off, ECH)], mbuf, lsem)
            pltpu.async_copy(dst_hbm.at[pl.ds(off, ECH)], ibuf, lsem)

        def step(s, mc, ic, lc, mn, inx, ln):
            pltpu.make_async_copy(m_hbm.at[pl.ds(0, ECH)], mc, lc).wait()
            pltpu.make_async_copy(dst_hbm.at[pl.ds(0, ECH)], ic, lc).wait()

            @pl.when(s < NCH - 1)
            def _():
                issue_loads(s + 1, mn, inx, ln)

            pltpu.sync_copy(mc, sh_agg.at[ic], add=True)

        issue_loads(0, mA, iA, lA_sem)

        @pl.loop(0, NCH)
        def _(s):
            @pl.when(s % 2 == 0)
            def _():
                step(s, mA, iA, lA_sem, mB, iB, lB_sem)

            @pl.when(s % 2 == 1)
            def _():
                step(s, mB, iB, lB_sem, mA, iA, lA_sem)

        plsc.subcore_barrier()
        r0 = sid * ROWS_PER_TILE
        pltpu.sync_copy(sh_agg.at[pl.ds(r0, ROWS_PER_TILE)],
                        agg_hbm.at[pl.ds(cid * NPAD + r0, ROWS_PER_TILE)])

    return agg_k(m, dst)


def _sc_cnt(dst):
    mesh = plsc.VectorSubcoreMesh(core_axis_name="c", subcore_axis_name="s")

    @functools.partial(
        pl.kernel,
        out_type=jax.ShapeDtypeStruct((NUM_SC * NPAD, D_F), _F32),
        mesh=mesh,
        scratch_types=[pltpu.VMEM((ECH, D_F), _F32),
                       pltpu.VMEM((ECH,), jnp.int32),
                       pltpu.VMEM((ECH,), jnp.int32),
                       pltpu.VMEM_SHARED((NPAD, D_F), _F32),
                       pltpu.SemaphoreType.DMA,
                       pltpu.SemaphoreType.DMA],
    )
    def cnt_k(dst_hbm, cnt_hbm, ones_v, iA, iB, sh_cnt, lA_sem, lB_sem):
        cid = lax.axis_index("c")
        sid = lax.axis_index("s")
        wid = sid * NUM_SC + cid
        base = wid * EPW

        @pl.loop(0, ECH)
        def _(r):
            @pl.loop(0, D_F, step=16)
            def _(c):
                ones_v.at[r, pl.ds(c, 16)][...] = jnp.zeros((16,), _F32)

        @pl.loop(0, ROWS_PER_TILE // ECH)
        def _(j):
            pltpu.sync_copy(ones_v, sh_cnt.at[pl.ds(sid * ROWS_PER_TILE + j * ECH, ECH)])

        @pl.loop(0, ECH)
        def _(r):
            @pl.loop(0, D_F, step=16)
            def _(c):
                ones_v.at[r, pl.ds(c, 16)][...] = jnp.full((16,), 1.0, _F32)

        plsc.subcore_barrier()

        def step(s, ic, lc, inx, ln):
            pltpu.make_async_copy(dst_hbm.at[pl.ds(0, ECH)], ic, lc).wait()

            @pl.when(s < NCH - 1)
            def _():
                pltpu.async_copy(dst_hbm.at[pl.ds(base + (s + 1) * ECH, ECH)], inx, ln)

            pltpu.sync_copy(ones_v, sh_cnt.at[ic], add=True)

        pltpu.async_copy(dst_hbm.at[pl.ds(base, ECH)], iA, lA_sem)

        @pl.loop(0, NCH)
        def _(s):
            @pl.when(s % 2 == 0)
            def _():
                step(s, iA, lA_sem, iB, lB_sem)

            @pl.when(s % 2 == 1)
            def _():
                step(s, iB, lB_sem, iA, lA_sem)

        plsc.subcore_barrier()
        r0 = sid * ROWS_PER_TILE
        pltpu.sync_copy(sh_cnt.at[pl.ds(r0, ROWS_PER_TILE)],
                        cnt_hbm.at[pl.ds(cid * NPAD + r0, ROWS_PER_TILE)])

    return cnt_k(dst)


def _node_body(h_ref, a0_ref, a1_ref, c0_ref, c1_ref, wn0h, wn0a, bn0, wn1, bn1,
               wn2, bn2, wo0, bo0, wo1, bo1, wo2, bo2, out_ref):
    cnt = c0_ref[...][:, 0:1] + c1_ref[...][:, 0:1]
    agg = (a0_ref[...] + a1_ref[...]) / jnp.maximum(cnt, 1.0)
    t = jax.nn.relu(_mm(h_ref[...], wn0h[...]) + _mm(agg, wn0a[...]) + bn0[...])
    t = jax.nn.relu(_mm(t, wn1[...]) + bn1[...])
    h2 = _mm(t, wn2[...]) + bn2[...]
    t = jax.nn.relu(_mm(h2, wo0[...]) + bo0[...])
    t = jax.nn.relu(_mm(t, wo1[...]) + bo1[...])
    out_ref[...] = _mm(t, wo2[...]) + bo2[...]


def _node_mlp(h, a0, a1, c0, c1, Wn0h, Wn0a, bn0, Wn1, bn1, Wn2, bn2,
              Wo0, bo0, Wo1, bo1, Wo2, bo2):
    full = lambda a: pl.BlockSpec(a.shape, lambda i: (0,) * a.ndim)
    row = lambda d: pl.BlockSpec((NBLK, d), lambda i: (i, 0))
    ws = (Wn0h, Wn0a, bn0, Wn1, bn1, Wn2, bn2, Wo0, bo0, Wo1, bo1, Wo2, bo2)
    return pl.pallas_call(
        _node_body,
        grid=(N // NBLK,),
        in_specs=[row(D_F), row(D_F), row(D_F), row(D_F), row(D_F)]
                 + [full(w) for w in ws],
        out_specs=pl.BlockSpec((NBLK, NCLS), lambda i: (i, 0)),
        out_shape=jax.ShapeDtypeStruct((N, NCLS), _F32),
    )(h, a0, a1, c0, c1, *ws)


def kernel(x, edge_index, edge_attr, We0, be0, We1, be1, We2, be2,
           Wi0, bi0, Wi1, bi1, Wi2, bi2, Wn0, bn0, Wn1, bn1, Wn2, bn2,
           Wo0, bo0, Wo1, bo1, Wo2, bo2):
    src = edge_index[0]
    dst = edge_index[1]
    r1 = lambda b: b.reshape(1, -1)

    cnt_flat = _sc_cnt(dst)
    h = _encode(x, We0, r1(be0), We1, r1(be1), We2, r1(be2))
    gs, gd = _sc_gather(h, src, dst)
    bf = jnp.bfloat16
    m = _edge_mlp(gs, gd, edge_attr,
                  Wi0[:D_F].astype(bf), Wi0[D_F:2 * D_F].astype(bf),
                  Wi0[2 * D_F:].astype(bf), r1(bi0),
                  Wi1.astype(bf), r1(bi1), Wi2.astype(bf), r1(bi2))
    agg_flat = _sc_agg(m, dst)
    aggp = agg_flat.reshape(NUM_SC, NPAD, D_F)[:, :N]
    cntp = cnt_flat.reshape(NUM_SC, NPAD, D_F)[:, :N]
    out = _node_mlp(h, aggp[0], aggp[1], cntp[0], cntp[1],
                    Wn0[:D_F], Wn0[D_F:], r1(bn0), Wn1, r1(bn1), Wn2, r1(bn2),
                    Wo0, r1(bo0), Wo1, r1(bo1), Wo2, r1(bo2))
    return out


# speedup vs baseline: 3.3303x; 1.0709x over previous
import functools

import jax
import jax.numpy as jnp
from jax import lax
from jax.experimental import pallas as pl
from jax.experimental.pallas import tpu as pltpu
from jax.experimental.pallas import tpu_sc as plsc

N = 10000
E = 320000
D_IN = 128
D_F = 128
D_H = 256
D_E = 16
NCLS = 10

NUM_SC = 2
NUM_TILES = 16
NW = NUM_SC * NUM_TILES
EPW = E // NW
ECH = 80
NCH = EPW // ECH
NPAD = 10240
ROWS_PER_TILE = NPAD // NUM_TILES

NBLK = 1000
EBLK = 2560

_F32 = jnp.float32


def _mm(a, b):
    return jnp.dot(a, b, preferred_element_type=jnp.float32)


def _encoder_body(x_ref, w0, b0, w1, b1, w2, b2, h_ref):
    t = jax.nn.relu(_mm(x_ref[...], w0[...]) + b0[...])
    t = jax.nn.relu(_mm(t, w1[...]) + b1[...])
    h_ref[...] = _mm(t, w2[...]) + b2[...]


def _encode(x, We0, be0, We1, be1, We2, be2):
    full = lambda a: pl.BlockSpec(a.shape, lambda i: (0,) * a.ndim)
    return pl.pallas_call(
        _encoder_body,
        grid=(N // NBLK,),
        in_specs=[pl.BlockSpec((NBLK, D_IN), lambda i: (i, 0)),
                  full(We0), full(be0), full(We1), full(be1), full(We2), full(be2)],
        out_specs=pl.BlockSpec((NBLK, D_F), lambda i: (i, 0)),
        out_shape=jax.ShapeDtypeStruct((N, D_F), _F32),
    )(x, We0, be0, We1, be1, We2, be2)


SCH = 200
GSUB = 5
GCH = SCH // GSUB
NSCH = EPW // SCH


def _sc_gather(h, src, dst):
    mesh = plsc.VectorSubcoreMesh(core_axis_name="c", subcore_axis_name="s")

    @functools.partial(
        pl.kernel,
        out_type=[jax.ShapeDtypeStruct((E, D_F), _F32),
                  jax.ShapeDtypeStruct((E, D_F), _F32)],
        mesh=mesh,
        scratch_types=[pltpu.VMEM((SCH,), jnp.int32),
                       pltpu.VMEM((SCH,), jnp.int32),
                       pltpu.VMEM((SCH, D_F), _F32),
                       pltpu.VMEM((SCH, D_F), _F32),
                       pltpu.VMEM((SCH,), jnp.int32),
                       pltpu.VMEM((SCH,), jnp.int32),
                       pltpu.VMEM((SCH, D_F), _F32),
                       pltpu.VMEM((SCH, D_F), _F32),
                       pltpu.SemaphoreType.DMA,
                       pltpu.SemaphoreType.DMA,
                       pltpu.SemaphoreType.DMA,
                       pltpu.SemaphoreType.DMA,
                       pltpu.SemaphoreType.DMA],
    )
    def k(h_hbm, src_hbm, dst_hbm, gs_hbm, gd_hbm,
          siA, diA, srA, drA, siB, diB, srB, drB,
          lA_sem, lB_sem, g_sem, wA_sem, wB_sem):
        wid = lax.axis_index("s") * NUM_SC + lax.axis_index("c")
        base = wid * EPW

        def issue_loads(s, si, di, lsem):
            off = base + s * SCH
            pltpu.async_copy(src_hbm.at[pl.ds(off, SCH)], si, lsem)
            pltpu.async_copy(dst_hbm.at[pl.ds(off, SCH)], di, lsem)

        def wait_loads(si, di, lsem):
            pltpu.make_async_copy(src_hbm.at[pl.ds(0, SCH)], si, lsem).wait()
            pltpu.make_async_copy(dst_hbm.at[pl.ds(0, SCH)], di, lsem).wait()

        def wait_writes(sr, dr, wsem):
            pltpu.make_async_copy(sr, gs_hbm.at[pl.ds(0, SCH)], wsem).wait()
            pltpu.make_async_copy(dr, gd_hbm.at[pl.ds(0, SCH)], wsem).wait()

        def step(s, si, di, sr, dr, lsem, wsem, nsi, ndi, nlsem):
            wait_loads(si, di, lsem)

            @pl.when(s > 1)
            def _():
                wait_writes(sr, dr, wsem)

            gathers = []
            for j in range(GSUB):
                sl = pl.ds(j * GCH, GCH)
                gathers.append(pltpu.async_copy(h_hbm.at[si.at[sl]], sr.at[sl], g_sem))
                gathers.append(pltpu.async_copy(h_hbm.at[di.at[sl]], dr.at[sl], g_sem))

            @pl.when(s < NSCH - 1)
            def _():
                issue_loads(s + 1, nsi, ndi, nlsem)

            for g in gathers:
                g.wait()
            off = base + s * SCH
            pltpu.async_copy(sr, gs_hbm.at[pl.ds(off, SCH)], wsem)
            pltpu.async_copy(dr, gd_hbm.at[pl.ds(off, SCH)], wsem)

        issue_loads(0, siA, diA, lA_sem)

        @pl.loop(0, NSCH)
        def _(s):
            @pl.when(s % 2 == 0)
            def _():
                step(s, siA, diA, srA, drA, lA_sem, wA_sem, siB, diB, lB_sem)

            @pl.when(s % 2 == 1)
            def _():
                step(s, siB, diB, srB, drB, lB_sem, wB_sem, siA, diA, lA_sem)

        wait_writes(srA, drA, wA_sem)
        wait_writes(srB, drB, wB_sem)

    return k(h, src, dst)


def _edge_body(gs_ref, gd_ref, ea_ref, w0s, w0d, w0e, b0, w1, b1, w2, b2, m_ref):
    bf = jnp.bfloat16
    t = (_mm(gs_ref[...].astype(bf), w0s[...]) + _mm(gd_ref[...].astype(bf), w0d[...])
         + _mm(ea_ref[...].astype(bf), w0e[...]) + b0[...])
    t = jax.nn.relu(t)
    t = jax.nn.relu(_mm(t.astype(bf), w1[...]) + b1[...])
    m_ref[...] = _mm(t.astype(bf), w2[...]) + b2[...]


def _edge_mlp(gs, gd, ea, Wi0s, Wi0d, Wi0e, bi0, Wi1, bi1, Wi2, bi2):
    full = lambda a: pl.BlockSpec(a.shape, lambda i: (0,) * a.ndim)
    return pl.pallas_call(
        _edge_body,
        grid=(E // EBLK,),
        in_specs=[pl.BlockSpec((EBLK, D_F), lambda i: (i, 0)),
                  pl.BlockSpec((EBLK, D_F), lambda i: (i, 0)),
                  pl.BlockSpec((EBLK, D_E), lambda i: (i, 0)),
                  full(Wi0s), full(Wi0d), full(Wi0e), full(bi0),
                  full(Wi1), full(bi1), full(Wi2), full(bi2)],
        out_specs=pl.BlockSpec((EBLK, D_F), lambda i: (i, 0)),
        out_shape=jax.ShapeDtypeStruct((E, D_F), _F32),
    )(gs, gd, ea, Wi0s, Wi0d, Wi0e, bi0, Wi1, bi1, Wi2, bi2)


def _sc_agg(m, dst):
    mesh = plsc.VectorSubcoreMesh(core_axis_name="c", subcore_axis_name="s")

    @functools.partial(
        pl.kernel,
        out_type=jax.ShapeDtypeStruct((NUM_SC * NPAD, D_F), _F32),
        mesh=mesh,
        scratch_types=[pltpu.VMEM((ECH, D_F), _F32),
                       pltpu.VMEM((ECH, D_F), _F32),
                       pltpu.VMEM((ECH,), jnp.int32),
                       pltpu.VMEM((ECH,), jnp.int32),
                       pltpu.VMEM_SHARED((NPAD, D_F), _F32),
                       pltpu.SemaphoreType.DMA,
                       pltpu.SemaphoreType.DMA],
    )
    def agg_k(m_hbm, dst_hbm, agg_hbm, mA, mB, iA, iB, sh_agg, lA_sem, lB_sem):
        cid = lax.axis_index("c")
        sid = lax.axis_index("s")
        wid = sid * NUM_SC + cid
        base = wid * EPW

        @pl.loop(0, ECH)
        def _(r):
            @pl.loop(0, D_F, step=16)
            def _(c):
                mA.at[r, pl.ds(c, 16)][...] = jnp.zeros((16,), _F32)

        @pl.loop(0, ROWS_PER_TILE // ECH)
        def _(j):
            pltpu.sync_copy(mA, sh_agg.at[pl.ds(sid * ROWS_PER_TILE + j * ECH, ECH)])

        plsc.subcore_barrier()

        def issue_loads(s, mbuf, ibuf, lsem):
            off = base + s * ECH
            pltpu.async_copy(m_hbm.at[pl.ds(off, ECH)], mbuf, lsem)
            pltpu.async_copy(dst_hbm.at[pl.ds(off, ECH)], ibuf, lsem)

        def step(s, mc, ic, lc, mn, inx, ln):
            pltpu.make_async_copy(m_hbm.at[pl.ds(0, ECH)], mc, lc).wait()
            pltpu.make_async_copy(dst_hbm.at[pl.ds(0, ECH)], ic, lc).wait()

            @pl.when(s < NCH - 1)
            def _():
                issue_loads(s + 1, mn, inx, ln)

            pltpu.sync_copy(mc, sh_agg.at[ic], add=True)

        issue_loads(0, mA, iA, lA_sem)

        @pl.loop(0, NCH)
        def _(s):
            @pl.when(s % 2 == 0)
            def _():
                step(s, mA, iA, lA_sem, mB, iB, lB_sem)

            @pl.when(s % 2 == 1)
            def _():
                step(s, mB, iB, lB_sem, mA, iA, lA_sem)

        plsc.subcore_barrier()
        r0 = sid * ROWS_PER_TILE
        pltpu.sync_copy(sh_agg.at[pl.ds(r0, ROWS_PER_TILE)],
                        agg_hbm.at[pl.ds(cid * NPAD + r0, ROWS_PER_TILE)])

    return agg_k(m, dst)


def _sc_cnt(dst):
    mesh = plsc.VectorSubcoreMesh(core_axis_name="c", subcore_axis_name="s")

    @functools.partial(
        pl.kernel,
        out_type=jax.ShapeDtypeStruct((NUM_SC * NPAD, D_F), _F32),
        mesh=mesh,
        scratch_types=[pltpu.VMEM((ECH, D_F), _F32),
                       pltpu.VMEM((ECH,), jnp.int32),
                       pltpu.VMEM((ECH,), jnp.int32),
                       pltpu.VMEM_SHARED((NPAD, D_F), _F32),
                       pltpu.SemaphoreType.DMA,
                       pltpu.SemaphoreType.DMA],
    )
    def cnt_k(dst_hbm, cnt_hbm, ones_v, iA, iB, sh_cnt, lA_sem, lB_sem):
        cid = lax.axis_index("c")
        sid = lax.axis_index("s")
        wid = sid * NUM_SC + cid
        base = wid * EPW

        @pl.loop(0, ECH)
        def _(r):
            @pl.loop(0, D_F, step=16)
            def _(c):
                ones_v.at[r, pl.ds(c, 16)][...] = jnp.zeros((16,), _F32)

        @pl.loop(0, ROWS_PER_TILE // ECH)
        def _(j):
            pltpu.sync_copy(ones_v, sh_cnt.at[pl.ds(sid * ROWS_PER_TILE + j * ECH, ECH)])

        @pl.loop(0, ECH)
        def _(r):
            @pl.loop(0, D_F, step=16)
            def _(c):
                ones_v.at[r, pl.ds(c, 16)][...] = jnp.full((16,), 1.0, _F32)

        plsc.subcore_barrier()

        def step(s, ic, lc, inx, ln):
            pltpu.make_async_copy(dst_hbm.at[pl.ds(0, ECH)], ic, lc).wait()

            @pl.when(s < NCH - 1)
            def _():
                pltpu.async_copy(dst_hbm.at[pl.ds(base + (s + 1) * ECH, ECH)], inx, ln)

            pltpu.sync_copy(ones_v, sh_cnt.at[ic], add=True)

        pltpu.async_copy(dst_hbm.at[pl.ds(base, ECH)], iA, lA_sem)

        @pl.loop(0, NCH)
        def _(s):
            @pl.when(s % 2 == 0)
            def _():
                step(s, iA, lA_sem, iB, lB_sem)

            @pl.when(s % 2 == 1)
            def _():
                step(s, iB, lB_sem, iA, lA_sem)

        plsc.subcore_barrier()
        r0 = sid * ROWS_PER_TILE
        pltpu.sync_copy(sh_cnt.at[pl.ds(r0, ROWS_PER_TILE)],
                        cnt_hbm.at[pl.ds(cid * NPAD + r0, ROWS_PER_TILE)])

    return cnt_k(dst)


def _node_body(h_ref, a0_ref, a1_ref, c0_ref, c1_ref, wn0h, wn0a, bn0, wn1, bn1,
               wn2, bn2, wo0, bo0, wo1, bo1, wo2, bo2, out_ref):
    cnt = c0_ref[...][:, 0:1] + c1_ref[...][:, 0:1]
    agg = (a0_ref[...] + a1_ref[...]) / jnp.maximum(cnt, 1.0)
    t = jax.nn.relu(_mm(h_ref[...], wn0h[...]) + _mm(agg, wn0a[...]) + bn0[...])
    t = jax.nn.relu(_mm(t, wn1[...]) + bn1[...])
    h2 = _mm(t, wn2[...]) + bn2[...]
    t = jax.nn.relu(_mm(h2, wo0[...]) + bo0[...])
    t = jax.nn.relu(_mm(t, wo1[...]) + bo1[...])
    out_ref[...] = _mm(t, wo2[...]) + bo2[...]


def _node_mlp(h, a0, a1, c0, c1, Wn0h, Wn0a, bn0, Wn1, bn1, Wn2, bn2,
              Wo0, bo0, Wo1, bo1, Wo2, bo2):
    full = lambda a: pl.BlockSpec(a.shape, lambda i: (0,) * a.ndim)
    row = lambda d: pl.BlockSpec((NBLK, d), lambda i: (i, 0))
    ws = (Wn0h, Wn0a, bn0, Wn1, bn1, Wn2, bn2, Wo0, bo0, Wo1, bo1, Wo2, bo2)
    return pl.pallas_call(
        _node_body,
        grid=(N // NBLK,),
        in_specs=[row(D_F), row(D_F), row(D_F), row(D_F), row(D_F)]
                 + [full(w) for w in ws],
        out_specs=pl.BlockSpec((NBLK, NCLS), lambda i: (i, 0)),
        out_shape=jax.ShapeDtypeStruct((N, NCLS), _F32),
    )(h, a0, a1, c0, c1, *ws)


def kernel(x, edge_index, edge_attr, We0, be0, We1, be1, We2, be2,
           Wi0, bi0, Wi1, bi1, Wi2, bi2, Wn0, bn0, Wn1, bn1, Wn2, bn2,
           Wo0, bo0, Wo1, bo1, Wo2, bo2):
    src = edge_index[0]
    dst = edge_index[1]
    r1 = lambda b: b.reshape(1, -1)

    cnt_flat = _sc_cnt(dst)
    h = _encode(x, We0, r1(be0), We1, r1(be1), We2, r1(be2))
    gs, gd = _sc_gather(h, src, dst)
    bf = jnp.bfloat16
    m = _edge_mlp(gs, gd, edge_attr,
                  Wi0[:D_F].astype(bf), Wi0[D_F:2 * D_F].astype(bf),
                  Wi0[2 * D_F:].astype(bf), r1(bi0),
                  Wi1.astype(bf), r1(bi1), Wi2.astype(bf), r1(bi2))
    agg_flat = _sc_agg(m, dst)
    aggp = agg_flat.reshape(NUM_SC, NPAD, D_F)[:, :N]
    cntp = cnt_flat.reshape(NUM_SC, NPAD, D_F)[:, :N]
    out = _node_mlp(h, aggp[0], aggp[1], cntp[0], cntp[1],
                    Wn0[:D_F], Wn0[D_F:], r1(bn0), Wn1, r1(bn1), Wn2, r1(bn2),
                    Wo0, r1(bo0), Wo1, r1(bo1), Wo2, r1(bo2))
    return out


# speedup vs baseline: 3.3884x; 1.0175x over previous
import functools

import jax
import jax.numpy as jnp
from jax import lax
from jax.experimental import pallas as pl
from jax.experimental.pallas import tpu as pltpu
from jax.experimental.pallas import tpu_sc as plsc

N = 10000
E = 320000
D_IN = 128
D_F = 128
D_H = 256
D_E = 16
NCLS = 10

NUM_SC = 2
NUM_TILES = 16
NW = NUM_SC * NUM_TILES
EPW = E // NW
ECH = 80
NCH = EPW // ECH
NPAD = 10240
ROWS_PER_TILE = NPAD // NUM_TILES

NBLK = 1000
EBLK = 4000

_F32 = jnp.float32


def _mm(a, b):
    return jnp.dot(a, b, preferred_element_type=jnp.float32)


def _encoder_body(x_ref, w0, b0, w1, b1, w2, b2, h_ref):
    t = jax.nn.relu(_mm(x_ref[...], w0[...]) + b0[...])
    t = jax.nn.relu(_mm(t, w1[...]) + b1[...])
    h_ref[...] = _mm(t, w2[...]) + b2[...]


def _encode(x, We0, be0, We1, be1, We2, be2):
    full = lambda a: pl.BlockSpec(a.shape, lambda i: (0,) * a.ndim)
    return pl.pallas_call(
        _encoder_body,
        grid=(N // NBLK,),
        in_specs=[pl.BlockSpec((NBLK, D_IN), lambda i: (i, 0)),
                  full(We0), full(be0), full(We1), full(be1), full(We2), full(be2)],
        out_specs=pl.BlockSpec((NBLK, D_F), lambda i: (i, 0)),
        out_shape=jax.ShapeDtypeStruct((N, D_F), _F32),
    )(x, We0, be0, We1, be1, We2, be2)


SCH = 200
GSUB = 5
GCH = SCH // GSUB
NSCH = EPW // SCH


def _sc_gather(h, src, dst):
    mesh = plsc.VectorSubcoreMesh(core_axis_name="c", subcore_axis_name="s")

    @functools.partial(
        pl.kernel,
        out_type=[jax.ShapeDtypeStruct((E, D_F), _F32),
                  jax.ShapeDtypeStruct((E, D_F), _F32)],
        mesh=mesh,
        scratch_types=[pltpu.VMEM((SCH,), jnp.int32),
                       pltpu.VMEM((SCH,), jnp.int32),
                       pltpu.VMEM((SCH, D_F), _F32),
                       pltpu.VMEM((SCH, D_F), _F32),
                       pltpu.VMEM((SCH,), jnp.int32),
                       pltpu.VMEM((SCH,), jnp.int32),
                       pltpu.VMEM((SCH, D_F), _F32),
                       pltpu.VMEM((SCH, D_F), _F32),
                       pltpu.SemaphoreType.DMA,
                       pltpu.SemaphoreType.DMA,
                       pltpu.SemaphoreType.DMA,
                       pltpu.SemaphoreType.DMA,
                       pltpu.SemaphoreType.DMA],
    )
    def k(h_hbm, src_hbm, dst_hbm, gs_hbm, gd_hbm,
          siA, diA, srA, drA, siB, diB, srB, drB,
          lA_sem, lB_sem, g_sem, wA_sem, wB_sem):
        wid = lax.axis_index("s") * NUM_SC + lax.axis_index("c")
        base = wid * EPW

        def issue_loads(s, si, di, lsem):
            off = base + s * SCH
            pltpu.async_copy(src_hbm.at[pl.ds(off, SCH)], si, lsem)
            pltpu.async_copy(dst_hbm.at[pl.ds(off, SCH)], di, lsem)

        def wait_loads(si, di, lsem):
            pltpu.make_async_copy(src_hbm.at[pl.ds(0, SCH)], si, lsem).wait()
            pltpu.make_async_copy(dst_hbm.at[pl.ds(0, SCH)], di, lsem).wait()

        def wait_writes(sr, dr, wsem):
            pltpu.make_async_copy(sr, gs_hbm.at[pl.ds(0, SCH)], wsem).wait()
            pltpu.make_async_copy(dr, gd_hbm.at[pl.ds(0, SCH)], wsem).wait()

        def step(s, si, di, sr, dr, lsem, wsem, nsi, ndi, nlsem):
            wait_loads(si, di, lsem)

            @pl.when(s > 1)
            def _():
                wait_writes(sr, dr, wsem)

            gathers = []
            for j in range(GSUB):
                sl = pl.ds(j * GCH, GCH)
                gathers.append(pltpu.async_copy(h_hbm.at[si.at[sl]], sr.at[sl], g_sem))
                gathers.append(pltpu.async_copy(h_hbm.at[di.at[sl]], dr.at[sl], g_sem))

            @pl.when(s < NSCH - 1)
            def _():
                issue_loads(s + 1, nsi, ndi, nlsem)

            for g in gathers:
                g.wait()
            off = base + s * SCH
            pltpu.async_copy(sr, gs_hbm.at[pl.ds(off, SCH)], wsem)
            pltpu.async_copy(dr, gd_hbm.at[pl.ds(off, SCH)], wsem)

        issue_loads(0, siA, diA, lA_sem)

        @pl.loop(0, NSCH)
        def _(s):
            @pl.when(s % 2 == 0)
            def _():
                step(s, siA, diA, srA, drA, lA_sem, wA_sem, siB, diB, lB_sem)

            @pl.when(s % 2 == 1)
            def _():
                step(s, siB, diB, srB, drB, lB_sem, wB_sem, siA, diA, lA_sem)

        wait_writes(srA, drA, wA_sem)
        wait_writes(srB, drB, wB_sem)

    return k(h, src, dst)


def _edge_body(gs_ref, gd_ref, ea_ref, w0s, w0d, w0e, b0, w1, b1, w2, b2, m_ref):
    bf = jnp.bfloat16
    t = (_mm(gs_ref[...].astype(bf), w0s[...]) + _mm(gd_ref[...].astype(bf), w0d[...])
         + _mm(ea_ref[...].astype(bf), w0e[...]) + b0[...])
    t = jax.nn.relu(t)
    t = jax.nn.relu(_mm(t.astype(bf), w1[...]) + b1[...])
    m_ref[...] = _mm(t.astype(bf), w2[...]) + b2[...]


def _edge_mlp(gs, gd, ea, Wi0s, Wi0d, Wi0e, bi0, Wi1, bi1, Wi2, bi2):
    full = lambda a: pl.BlockSpec(a.shape, lambda i: (0,) * a.ndim)
    return pl.pallas_call(
        _edge_body,
        grid=(E // EBLK,),
        in_specs=[pl.BlockSpec((EBLK, D_F), lambda i: (i, 0)),
                  pl.BlockSpec((EBLK, D_F), lambda i: (i, 0)),
                  pl.BlockSpec((EBLK, D_E), lambda i: (i, 0)),
                  full(Wi0s), full(Wi0d), full(Wi0e), full(bi0),
                  full(Wi1), full(bi1), full(Wi2), full(bi2)],
        out_specs=pl.BlockSpec((EBLK, D_F), lambda i: (i, 0)),
        out_shape=jax.ShapeDtypeStruct((E, D_F), _F32),
    )(gs, gd, ea, Wi0s, Wi0d, Wi0e, bi0, Wi1, bi1, Wi2, bi2)


def _sc_agg(m, dst):
    mesh = plsc.VectorSubcoreMesh(core_axis_name="c", subcore_axis_name="s")

    @functools.partial(
        pl.kernel,
        out_type=jax.ShapeDtypeStruct((NUM_SC * NPAD, D_F), _F32),
        mesh=mesh,
        scratch_types=[pltpu.VMEM((ECH, D_F), _F32),
                       pltpu.VMEM((ECH, D_F), _F32),
                       pltpu.VMEM((ECH,), jnp.int32),
                       pltpu.VMEM((ECH,), jnp.int32),
                       pltpu.VMEM_SHARED((NPAD, D_F), _F32),
                       pltpu.SemaphoreType.DMA,
                       pltpu.SemaphoreType.DMA],
    )
    def agg_k(m_hbm, dst_hbm, agg_hbm, mA, mB, iA, iB, sh_agg, lA_sem, lB_sem):
        cid = lax.axis_index("c")
        sid = lax.axis_index("s")
        wid = sid * NUM_SC + cid
        base = wid * EPW

        @pl.loop(0, ECH)
        def _(r):
            @pl.loop(0, D_F, step=16)
            def _(c):
                mA.at[r, pl.ds(c, 16)][...] = jnp.zeros((16,), _F32)

        @pl.loop(0, ROWS_PER_TILE // ECH)
        def _(j):
            pltpu.sync_copy(mA, sh_agg.at[pl.ds(sid * ROWS_PER_TILE + j * ECH, ECH)])

        plsc.subcore_barrier()

        def issue_loads(s, mbuf, ibuf, lsem):
            off = base + s * ECH
            pltpu.async_copy(m_hbm.at[pl.ds(off, ECH)], mbuf, lsem)
            pltpu.async_copy(dst_hbm.at[pl.ds(off, ECH)], ibuf, lsem)

        def step(s, mc, ic, lc, mn, inx, ln):
            pltpu.make_async_copy(m_hbm.at[pl.ds(0, ECH)], mc, lc).wait()
            pltpu.make_async_copy(dst_hbm.at[pl.ds(0, ECH)], ic, lc).wait()

            @pl.when(s < NCH - 1)
            def _():
                issue_loads(s + 1, mn, inx, ln)

            pltpu.sync_copy(mc, sh_agg.at[ic], add=True)

        issue_loads(0, mA, iA, lA_sem)

        @pl.loop(0, NCH)
        def _(s):
            @pl.when(s % 2 == 0)
            def _():
                step(s, mA, iA, lA_sem, mB, iB, lB_sem)

            @pl.when(s % 2 == 1)
            def _():
                step(s, mB, iB, lB_sem, mA, iA, lA_sem)

        plsc.subcore_barrier()
        r0 = sid * ROWS_PER_TILE
        pltpu.sync_copy(sh_agg.at[pl.ds(r0, ROWS_PER_TILE)],
                        agg_hbm.at[pl.ds(cid * NPAD + r0, ROWS_PER_TILE)])

    return agg_k(m, dst)


def _sc_cnt(dst):
    mesh = plsc.VectorSubcoreMesh(core_axis_name="c", subcore_axis_name="s")

    @functools.partial(
        pl.kernel,
        out_type=jax.ShapeDtypeStruct((NUM_SC * NPAD, D_F), _F32),
        mesh=mesh,
        scratch_types=[pltpu.VMEM((ECH, D_F), _F32),
                       pltpu.VMEM((ECH,), jnp.int32),
                       pltpu.VMEM((ECH,), jnp.int32),
                       pltpu.VMEM_SHARED((NPAD, D_F), _F32),
                       pltpu.SemaphoreType.DMA,
                       pltpu.SemaphoreType.DMA],
    )
    def cnt_k(dst_hbm, cnt_hbm, ones_v, iA, iB, sh_cnt, lA_sem, lB_sem):
        cid = lax.axis_index("c")
        sid = lax.axis_index("s")
        wid = sid * NUM_SC + cid
        base = wid * EPW

        @pl.loop(0, ECH)
        def _(r):
            @pl.loop(0, D_F, step=16)
            def _(c):
                ones_v.at[r, pl.ds(c, 16)][...] = jnp.zeros((16,), _F32)

        @pl.loop(0, ROWS_PER_TILE // ECH)
        def _(j):
            pltpu.sync_copy(ones_v, sh_cnt.at[pl.ds(sid * ROWS_PER_TILE + j * ECH, ECH)])

        @pl.loop(0, ECH)
        def _(r):
            @pl.loop(0, D_F, step=16)
            def _(c):
                ones_v.at[r, pl.ds(c, 16)][...] = jnp.full((16,), 1.0, _F32)

        plsc.subcore_barrier()

        def step(s, ic, lc, inx, ln):
            pltpu.make_async_copy(dst_hbm.at[pl.ds(0, ECH)], ic, lc).wait()

            @pl.when(s < NCH - 1)
            def _():
                pltpu.async_copy(dst_hbm.at[pl.ds(base + (s + 1) * ECH, ECH)], inx, ln)

            pltpu.sync_copy(ones_v, sh_cnt.at[ic], add=True)

        pltpu.async_copy(dst_hbm.at[pl.ds(base, ECH)], iA, lA_sem)

        @pl.loop(0, NCH)
        def _(s):
            @pl.when(s % 2 == 0)
            def _():
                step(s, iA, lA_sem, iB, lB_sem)

            @pl.when(s % 2 == 1)
            def _():
                step(s, iB, lB_sem, iA, lA_sem)

        plsc.subcore_barrier()
        r0 = sid * ROWS_PER_TILE
        pltpu.sync_copy(sh_cnt.at[pl.ds(r0, ROWS_PER_TILE)],
                        cnt_hbm.at[pl.ds(cid * NPAD + r0, ROWS_PER_TILE)])

    return cnt_k(dst)


def _node_body(h_ref, a0_ref, a1_ref, c0_ref, c1_ref, wn0h, wn0a, bn0, wn1, bn1,
               wn2, bn2, wo0, bo0, wo1, bo1, wo2, bo2, out_ref):
    cnt = c0_ref[...][:, 0:1] + c1_ref[...][:, 0:1]
    agg = (a0_ref[...] + a1_ref[...]) / jnp.maximum(cnt, 1.0)
    t = jax.nn.relu(_mm(h_ref[...], wn0h[...]) + _mm(agg, wn0a[...]) + bn0[...])
    t = jax.nn.relu(_mm(t, wn1[...]) + bn1[...])
    h2 = _mm(t, wn2[...]) + bn2[...]
    t = jax.nn.relu(_mm(h2, wo0[...]) + bo0[...])
    t = jax.nn.relu(_mm(t, wo1[...]) + bo1[...])
    out_ref[...] = _mm(t, wo2[...]) + bo2[...]


def _node_mlp(h, a0, a1, c0, c1, Wn0h, Wn0a, bn0, Wn1, bn1, Wn2, bn2,
              Wo0, bo0, Wo1, bo1, Wo2, bo2):
    full = lambda a: pl.BlockSpec(a.shape, lambda i: (0,) * a.ndim)
    row = lambda d: pl.BlockSpec((NBLK, d), lambda i: (i, 0))
    ws = (Wn0h, Wn0a, bn0, Wn1, bn1, Wn2, bn2, Wo0, bo0, Wo1, bo1, Wo2, bo2)
    return pl.pallas_call(
        _node_body,
        grid=(N // NBLK,),
        in_specs=[row(D_F), row(D_F), row(D_F), row(D_F), row(D_F)]
                 + [full(w) for w in ws],
        out_specs=pl.BlockSpec((NBLK, NCLS), lambda i: (i, 0)),
        out_shape=jax.ShapeDtypeStruct((N, NCLS), _F32),
    )(h, a0, a1, c0, c1, *ws)


def kernel(x, edge_index, edge_attr, We0, be0, We1, be1, We2, be2,
           Wi0, bi0, Wi1, bi1, Wi2, bi2, Wn0, bn0, Wn1, bn1, Wn2, bn2,
           Wo0, bo0, Wo1, bo1, Wo2, bo2):
    src = edge_index[0]
    dst = edge_index[1]
    r1 = lambda b: b.reshape(1, -1)

    cnt_flat = _sc_cnt(dst)
    h = _encode(x, We0, r1(be0), We1, r1(be1), We2, r1(be2))
    gs, gd = _sc_gather(h, src, dst)
    bf = jnp.bfloat16
    m = _edge_mlp(gs, gd, edge_attr,
                  Wi0[:D_F].astype(bf), Wi0[D_F:2 * D_F].astype(bf),
                  Wi0[2 * D_F:].astype(bf), r1(bi0),
                  Wi1.astype(bf), r1(bi1), Wi2.astype(bf), r1(bi2))
    agg_flat = _sc_agg(m, dst)
    aggp = agg_flat.reshape(NUM_SC, NPAD, D_F)[:, :N]
    cntp = cnt_flat.reshape(NUM_SC, NPAD, D_F)[:, :N]
    out = _node_mlp(h, aggp[0], aggp[1], cntp[0], cntp[1],
                    Wn0[:D_F], Wn0[D_F:], r1(bn0), Wn1, r1(bn1), Wn2, r1(bn2),
                    Wo0, r1(bo0), Wo1, r1(bo1), Wo2, r1(bo2))
    return out
